# baseline (speedup 1.0000x reference)
import functools

import jax
import jax.numpy as jnp
from jax import lax
from jax.experimental import pallas as pl
from jax.experimental.pallas import tpu as pltpu
from jax.experimental.pallas import tpu_sc as plsc

NN = 10000
NE = 160000
D = 128
DH = 384

NC = 2
NS = 16
NW = NC * NS
CH = 128

E_BLK = 2000
R_BLK = 2000

_SC_MESH = plsc.VectorSubcoreMesh(core_axis_name="c", subcore_axis_name="s")


def _gather_rows(table, idx):
    B = idx.shape[0]
    n_chunks = B // CH
    max_nj = -(-n_chunks // NW)

    @functools.partial(
        pl.kernel,
        out_type=jax.ShapeDtypeStruct((B, D), jnp.float32),
        mesh=_SC_MESH,
        scratch_types=[
            pltpu.VMEM((CH,), jnp.int32),
            pltpu.VMEM((CH, D), jnp.float32),
            pltpu.SemaphoreType.DMA,
        ],
    )
    def k(table_hbm, idx_hbm, out_hbm, idx_v, rows_v, sem):
        wid = lax.axis_index("s") * NC + lax.axis_index("c")

        @pl.loop(0, max_nj)
        def _(j):
            c = j * NW + wid

            @pl.when(c < n_chunks)
            def _():
                off = c * CH
                pltpu.sync_copy(idx_hbm.at[pl.ds(off, CH)], idx_v)
                pltpu.async_copy(table_hbm.at[idx_v], rows_v, sem).wait()
                pltpu.sync_copy(rows_v, out_hbm.at[pl.ds(off, CH)])

    return k(table, idx)


def _per_sub_slices(sid, fn):
    rps = 624

    @pl.when(sid < NS - 1)
    def _():
        fn(sid * rps, rps)

    @pl.when(sid == NS - 1)
    def _():
        fn((NS - 1) * rps, NN - (NS - 1) * rps)


def _segment_sum(m, dst, zeros_d):
    n_chunks = NE // CH
    max_nj = -(-n_chunks // NW)

    @functools.partial(
        pl.kernel,
        out_type=jax.ShapeDtypeStruct((NC, NN, D), jnp.float32),
        mesh=_SC_MESH,
        scratch_types=[
            pltpu.VMEM((CH,), jnp.int32),
            pltpu.VMEM((CH, D), jnp.float32),
            pltpu.VMEM_SHARED((NN, D), jnp.float32),
            pltpu.SemaphoreType.DMA,
        ],
    )
    def k(m_hbm, dst_hbm, z_d_hbm, agg_hbm, idx_v, rows_v, acc_sh, sem):
        cid = lax.axis_index("c")
        sid = lax.axis_index("s")
        wid = sid * NC + cid

        _per_sub_slices(sid, lambda o, n: pltpu.sync_copy(
            z_d_hbm.at[pl.ds(o, n)], acc_sh.at[pl.ds(o, n)]))
        plsc.subcore_barrier()

        @pl.loop(0, max_nj)
        def _(j):
            c = j * NW + wid

            @pl.when(c < n_chunks)
            def _():
                off = c * CH
                pltpu.sync_copy(dst_hbm.at[pl.ds(off, CH)], idx_v)
                pltpu.sync_copy(m_hbm.at[pl.ds(off, CH)], rows_v)
                pltpu.sync_copy(rows_v, acc_sh.at[idx_v], add=True)

        plsc.subcore_barrier()
        _per_sub_slices(sid, lambda o, n: pltpu.sync_copy(
            acc_sh.at[pl.ds(o, n)], agg_hbm.at[cid, pl.ds(o, n)]))

    return k(m, dst, zeros_d)


def _segment_count(dst, zeros_d, ones_d):
    n_chunks = NE // CH
    max_nj = -(-n_chunks // NW)

    @functools.partial(
        pl.kernel,
        out_type=jax.ShapeDtypeStruct((NC, NN, D), jnp.float32),
        mesh=_SC_MESH,
        scratch_types=[
            pltpu.VMEM((CH,), jnp.int32),
            pltpu.VMEM((CH, D), jnp.float32),
            pltpu.VMEM_SHARED((NN, D), jnp.float32),
            pltpu.SemaphoreType.DMA,
        ],
    )
    def k(dst_hbm, z_d_hbm, ones_hbm, deg_hbm, idx_v, ones_v, acc_sh, sem):
        cid = lax.axis_index("c")
        sid = lax.axis_index("s")
        wid = sid * NC + cid

        _per_sub_slices(sid, lambda o, n: pltpu.sync_copy(
            z_d_hbm.at[pl.ds(o, n)], acc_sh.at[pl.ds(o, n)]))
        pltpu.sync_copy(ones_hbm, ones_v)
        plsc.subcore_barrier()

        @pl.loop(0, max_nj)
        def _(j):
            c = j * NW + wid

            @pl.when(c < n_chunks)
            def _():
                pltpu.sync_copy(dst_hbm.at[pl.ds(c * CH, CH)], idx_v)
                pltpu.sync_copy(ones_v, acc_sh.at[idx_v], add=True)

        plsc.subcore_barrier()
        _per_sub_slices(sid, lambda o, n: pltpu.sync_copy(
            acc_sh.at[pl.ds(o, n)], deg_hbm.at[cid, pl.ds(o, n)]))

    return k(dst, zeros_d, ones_d)


def _dot(a, w):
    return jnp.dot(a.astype(jnp.bfloat16), w,
                   preferred_element_type=jnp.float32)


def _chain(a, ws_ref, bs_ref, n_hidden):
    for i in range(n_hidden):
        a = _dot(a, ws_ref[i])
        a = a + bs_ref[i + 1]
        if i < n_hidden - 1:
            a = jnp.maximum(a, 0.0)
    return a


def _msg_body(gs_ref, gd_ref, ang_ref, w0_ref, ws_ref, bs_ref, m_ref):
    h = jnp.concatenate([gs_ref[...], gd_ref[...]], axis=1)
    a = _dot(h, w0_ref[...]) + bs_ref[0]
    a = jnp.maximum(a, 0.0)
    a = _chain(a, ws_ref, bs_ref, 5)
    m_ref[...] = a * ang_ref[...]


def _msg_mlp(gs, gd, ang, w0, ws, bs):
    grid = NE // E_BLK
    return pl.pallas_call(
        _msg_body,
        grid=(grid,),
        in_specs=[
            pl.BlockSpec((E_BLK, D), lambda i: (i, 0)),
            pl.BlockSpec((E_BLK, D), lambda i: (i, 0)),
            pl.BlockSpec((E_BLK, 1), lambda i: (i, 0)),
            pl.BlockSpec((2 * D, D), lambda i: (0, 0)),
            pl.BlockSpec((5, D, D), lambda i: (0, 0, 0)),
            pl.BlockSpec((6, D), lambda i: (0, 0)),
        ],
        out_specs=pl.BlockSpec((E_BLK, D), lambda i: (i, 0)),
        out_shape=jax.ShapeDtypeStruct((NE, D), jnp.float32),
    )(gs, gd, ang, w0, ws, bs)


def _upd_body(x_ref, agg_ref, deg_ref, w_ref, b_ref, o_ref):
    agg = agg_ref[0] + agg_ref[1]
    deg = deg_ref[0, :, 0:1] + deg_ref[1, :, 0:1]
    agg = agg / jnp.maximum(deg, 1.0)
    h = jnp.concatenate([x_ref[...], agg], axis=1)
    o = _dot(h, w_ref[...]) + b_ref[...]
    o_ref[...] = jnp.maximum(o, 0.0)


def _upd_mlp(x, agg_p, deg_p, w, b):
    grid = NN // R_BLK
    return pl.pallas_call(
        _upd_body,
        grid=(grid,),
        in_specs=[
            pl.BlockSpec((R_BLK, D), lambda i: (i, 0)),
            pl.BlockSpec((NC, R_BLK, D), lambda i: (0, i, 0)),
            pl.BlockSpec((NC, R_BLK, D), lambda i: (0, i, 0)),
            pl.BlockSpec((2 * D, D), lambda i: (0, 0)),
            pl.BlockSpec((1, D), lambda i: (0, 0)),
        ],
        out_specs=pl.BlockSpec((R_BLK, D), lambda i: (i, 0)),
        out_shape=jax.ShapeDtypeStruct((NN, D), jnp.float32),
    )(x, agg_p, deg_p, w, b)


def _edge_stack(gs_ref, gd_ref, w0_ref, ws_ref, bs_ref):
    hf = jnp.concatenate([gs_ref[...], gd_ref[...]], axis=1)
    hr = jnp.concatenate([gd_ref[...], gs_ref[...]], axis=1)
    h = jnp.concatenate([hf, hr], axis=0)
    a = _dot(h, w0_ref[...]) + bs_ref[0]
    a = jnp.maximum(a, 0.0)
    a = _chain(a, ws_ref, bs_ref, 5)
    return a[:E_BLK], a[E_BLK:]


def _edge1_body(gs_ref, gd_ref, w0_ref, ws_ref, bs_ref, ef_ref, sl_ref):
    e1, e2 = _edge_stack(gs_ref, gd_ref, w0_ref, ws_ref, bs_ref)

    @pl.when(pl.program_id(0) == 0)
    def _():
        sl_ref[...] = jnp.zeros((1, 1), jnp.float32)

    d = e1 - e2
    sl_ref[...] += jnp.sum(d * d).reshape(1, 1)
    ef_ref[...] = 0.5 * (e1 + e2)


def _edge1_mlp(gs, gd, w0, ws, bs):
    grid = NE // E_BLK
    return pl.pallas_call(
        _edge1_body,
        grid=(grid,),
        in_specs=[
            pl.BlockSpec((E_BLK, D), lambda i: (i, 0)),
            pl.BlockSpec((E_BLK, D), lambda i: (i, 0)),
            pl.BlockSpec((2 * D, DH), lambda i: (0, 0)),
            pl.BlockSpec((5, DH, DH), lambda i: (0, 0, 0)),
            pl.BlockSpec((6, DH), lambda i: (0, 0)),
        ],
        out_specs=[
            pl.BlockSpec((E_BLK, DH), lambda i: (i, 0)),
            pl.BlockSpec((1, 1), lambda i: (0, 0)),
        ],
        out_shape=[
            jax.ShapeDtypeStruct((NE, DH), jnp.float32),
            jax.ShapeDtypeStruct((1, 1), jnp.float32),
        ],
    )(gs, gd, w0, ws, bs)


def _edge2_body(gs_ref, gd_ref, ef1_ref, w0_ref, ws_ref, bs_ref,
                wfe_ref, wfp_ref, fb_ref, ef_ref, sl_ref):
    e1, e2 = _edge_stack(gs_ref, gd_ref, w0_ref, ws_ref, bs_ref)

    @pl.when(pl.program_id(0) == 0)
    def _():
        sl_ref[...] = jnp.zeros((1, 1), jnp.float32)

    d = e1 - e2
    sl_ref[...] += jnp.sum(d * d).reshape(1, 1)
    e = 0.5 * (e1 + e2)
    o = _dot(e, wfe_ref[...])
    o += _dot(ef1_ref[...], wfp_ref[...])
    ef_ref[...] = o + fb_ref[...]


def _edge2_mlp(gs, gd, ef1, w0, ws, bs, wfe, wfp, fb):
    grid = NE // E_BLK
    return pl.pallas_call(
        _edge2_body,
        grid=(grid,),
        in_specs=[
            pl.BlockSpec((E_BLK, D), lambda i: (i, 0)),
            pl.BlockSpec((E_BLK, D), lambda i: (i, 0)),
            pl.BlockSpec((E_BLK, DH), lambda i: (i, 0)),
            pl.BlockSpec((2 * D, DH), lambda i: (0, 0)),
            pl.BlockSpec((5, DH, DH), lambda i: (0, 0, 0)),
            pl.BlockSpec((6, DH), lambda i: (0, 0)),
            pl.BlockSpec((DH, D), lambda i: (0, 0)),
            pl.BlockSpec((DH, D), lambda i: (0, 0)),
            pl.BlockSpec((1, D), lambda i: (0, 0)),
        ],
        out_specs=[
            pl.BlockSpec((E_BLK, D), lambda i: (i, 0)),
            pl.BlockSpec((1, 1), lambda i: (0, 0)),
        ],
        out_shape=[
            jax.ShapeDtypeStruct((NE, D), jnp.float32),
            jax.ShapeDtypeStruct((1, 1), jnp.float32),
        ],
    )(gs, gd, ef1, w0, ws, bs, wfe, wfp, fb)


def _stack_mlp(p):
    w0 = p[0][0].astype(jnp.bfloat16)
    ws = jnp.stack([w for (w, _) in p[1:]]).astype(jnp.bfloat16)
    bs = jnp.stack([b for (_, b) in p])
    return w0, ws, bs


def kernel(node_features, edge_index, angles, gt_edges, params):
    del gt_edges
    src = edge_index[0]
    dst = edge_index[1]
    idx_all = jnp.concatenate([src, dst])
    ang = angles.reshape(NE, 1)

    m1w0, m1ws, m1bs = _stack_mlp(params["nc1"]["msg"])
    m2w0, m2ws, m2bs = _stack_mlp(params["nc2"]["msg"])
    e1w0, e1ws, e1bs = _stack_mlp(params["ec1"]["edge"])
    e2w0, e2ws, e2bs = _stack_mlp(params["ec2"]["edge"])
    u1w, u1b = params["nc1"]["upd"][0]
    u2w, u2b = params["nc2"]["upd"][0]
    fw, fb = params["ec2"]["fuse"][0]
    u1w = u1w.astype(jnp.bfloat16)
    u2w = u2w.astype(jnp.bfloat16)
    fw = fw.astype(jnp.bfloat16)
    u1b = u1b.reshape(1, D)
    u2b = u2b.reshape(1, D)
    fb = fb.reshape(1, D)
    wfe, wfp = fw[:DH], fw[DH:]

    zeros_d = jnp.zeros((NN, D), jnp.float32)
    ones_d = jnp.ones((CH, D), jnp.float32)

    x0 = node_features

    dp = _segment_count(dst, zeros_d, ones_d)

    g0 = _gather_rows(x0, idx_all)
    m1 = _msg_mlp(g0[:NE], g0[NE:], ang, m1w0, m1ws, m1bs)
    a1 = _segment_sum(m1, dst, zeros_d)
    x1 = _upd_mlp(x0, a1, dp, u1w, u1b)

    g1 = _gather_rows(x1, idx_all)
    g1s, g1d = g1[:NE], g1[NE:]

    m2 = _msg_mlp(g1s, g1d, ang, m2w0, m2ws, m2bs)
    a2 = _segment_sum(m2, dst, zeros_d)
    x2 = _upd_mlp(x1, a2, dp, u2w, u2b)
    g2 = _gather_rows(x2, idx_all)

    ef1, sl1 = _edge1_mlp(g1s, g1d, e1w0, e1ws, e1bs)

    ef, sl2 = _edge2_mlp(g2[:NE], g2[NE:], ef1, e2w0, e2ws, e2bs, wfe, wfp, fb)

    side_loss = (sl1[0, 0] + sl2[0, 0]) / (2.0 * NE * DH)
    return ef, side_loss


# speedup vs baseline: 1.0014x; 1.0014x over previous
import functools

import jax
import jax.numpy as jnp
from jax import lax
from jax.experimental import pallas as pl
from jax.experimental.pallas import tpu as pltpu
from jax.experimental.pallas import tpu_sc as plsc

NN = 10000
NE = 160000
D = 128
DH = 384

NC = 2
NS = 16
NW = NC * NS
CH = 128

E_BLK = 2000
R_BLK = 2000

_SC_MESH = plsc.VectorSubcoreMesh(core_axis_name="c", subcore_axis_name="s")


def _gather_rows(table, idx):
    B = idx.shape[0]
    n_chunks = B // CH
    max_nj = -(-n_chunks // NW)
    dt = table.dtype
    W = table.shape[1]

    @functools.partial(
        pl.kernel,
        out_type=jax.ShapeDtypeStruct((B, W), dt),
        mesh=_SC_MESH,
        scratch_types=[
            pltpu.VMEM((CH,), jnp.int32),
            pltpu.VMEM((CH, W), dt),
            pltpu.SemaphoreType.DMA,
        ],
    )
    def k(table_hbm, idx_hbm, out_hbm, idx_v, rows_v, sem):
        wid = lax.axis_index("s") * NC + lax.axis_index("c")

        @pl.loop(0, max_nj)
        def _(j):
            c = j * NW + wid

            @pl.when(c < n_chunks)
            def _():
                off = c * CH
                pltpu.sync_copy(idx_hbm.at[pl.ds(off, CH)], idx_v)
                pltpu.async_copy(table_hbm.at[idx_v], rows_v, sem).wait()
                pltpu.sync_copy(rows_v, out_hbm.at[pl.ds(off, CH)])

    return k(table, idx)


def _per_sub_slices(sid, fn):
    rps = 624

    @pl.when(sid < NS - 1)
    def _():
        fn(sid * rps, rps)

    @pl.when(sid == NS - 1)
    def _():
        fn((NS - 1) * rps, NN - (NS - 1) * rps)


def _segment_sum(m, dst, zeros_d):
    n_chunks = NE // CH
    max_nj = -(-n_chunks // NW)

    @functools.partial(
        pl.kernel,
        out_type=jax.ShapeDtypeStruct((NC, NN, D), jnp.float32),
        mesh=_SC_MESH,
        scratch_types=[
            pltpu.VMEM((CH,), jnp.int32),
            pltpu.VMEM((CH, D), jnp.float32),
            pltpu.VMEM_SHARED((NN, D), jnp.float32),
            pltpu.SemaphoreType.DMA,
        ],
    )
    def k(m_hbm, dst_hbm, z_d_hbm, agg_hbm, idx_v, rows_v, acc_sh, sem):
        cid = lax.axis_index("c")
        sid = lax.axis_index("s")
        wid = sid * NC + cid

        _per_sub_slices(sid, lambda o, n: pltpu.sync_copy(
            z_d_hbm.at[pl.ds(o, n)], acc_sh.at[pl.ds(o, n)]))
        plsc.subcore_barrier()

        @pl.loop(0, max_nj)
        def _(j):
            c = j * NW + wid

            @pl.when(c < n_chunks)
            def _():
                off = c * CH
                pltpu.sync_copy(dst_hbm.at[pl.ds(off, CH)], idx_v)
                pltpu.sync_copy(m_hbm.at[pl.ds(off, CH)], rows_v)
                pltpu.sync_copy(rows_v, acc_sh.at[idx_v], add=True)

        plsc.subcore_barrier()
        _per_sub_slices(sid, lambda o, n: pltpu.sync_copy(
            acc_sh.at[pl.ds(o, n)], agg_hbm.at[cid, pl.ds(o, n)]))

    return k(m, dst, zeros_d)


def _segment_count(dst, zeros_d, ones_d):
    n_chunks = NE // CH
    max_nj = -(-n_chunks // NW)

    @functools.partial(
        pl.kernel,
        out_type=jax.ShapeDtypeStruct((NC, NN, D), jnp.float32),
        mesh=_SC_MESH,
        scratch_types=[
            pltpu.VMEM((CH,), jnp.int32),
            pltpu.VMEM((CH, D), jnp.float32),
            pltpu.VMEM_SHARED((NN, D), jnp.float32),
            pltpu.SemaphoreType.DMA,
        ],
    )
    def k(dst_hbm, z_d_hbm, ones_hbm, deg_hbm, idx_v, ones_v, acc_sh, sem):
        cid = lax.axis_index("c")
        sid = lax.axis_index("s")
        wid = sid * NC + cid

        _per_sub_slices(sid, lambda o, n: pltpu.sync_copy(
            z_d_hbm.at[pl.ds(o, n)], acc_sh.at[pl.ds(o, n)]))
        pltpu.sync_copy(ones_hbm, ones_v)
        plsc.subcore_barrier()

        @pl.loop(0, max_nj)
        def _(j):
            c = j * NW + wid

            @pl.when(c < n_chunks)
            def _():
                pltpu.sync_copy(dst_hbm.at[pl.ds(c * CH, CH)], idx_v)
                pltpu.sync_copy(ones_v, acc_sh.at[idx_v], add=True)

        plsc.subcore_barrier()
        _per_sub_slices(sid, lambda o, n: pltpu.sync_copy(
            acc_sh.at[pl.ds(o, n)], deg_hbm.at[cid, pl.ds(o, n)]))

    return k(dst, zeros_d, ones_d)


def _chain(a, ws_ref, bs_ref, n_hidden):
    for i in range(n_hidden - 1):
        a = jnp.dot(a, ws_ref[i], preferred_element_type=jnp.float32)
        a = jnp.maximum(a + bs_ref[i + 1], 0.0).astype(jnp.bfloat16)
    a = jnp.dot(a, ws_ref[n_hidden - 1], preferred_element_type=jnp.float32)
    return a + bs_ref[n_hidden]


def _msg_body(gs_ref, gd_ref, ang_ref, w0_ref, ws_ref, bs_ref, m_ref):
    h = jnp.concatenate([gs_ref[...], gd_ref[...]], axis=1)
    a = jnp.dot(h, w0_ref[...], preferred_element_type=jnp.float32)
    a = jnp.maximum(a + bs_ref[0], 0.0).astype(jnp.bfloat16)
    a = _chain(a, ws_ref, bs_ref, 5)
    m_ref[...] = a * ang_ref[...]


def _msg_mlp(gs, gd, ang, w0, ws, bs):
    grid = NE // E_BLK
    return pl.pallas_call(
        _msg_body,
        grid=(grid,),
        in_specs=[
            pl.BlockSpec((E_BLK, D), lambda i: (i, 0)),
            pl.BlockSpec((E_BLK, D), lambda i: (i, 0)),
            pl.BlockSpec((E_BLK, 1), lambda i: (i, 0)),
            pl.BlockSpec((2 * D, D), lambda i: (0, 0)),
            pl.BlockSpec((5, D, D), lambda i: (0, 0, 0)),
            pl.BlockSpec((6, D), lambda i: (0, 0)),
        ],
        out_specs=pl.BlockSpec((E_BLK, D), lambda i: (i, 0)),
        out_shape=jax.ShapeDtypeStruct((NE, D), jnp.float32),
    )(gs, gd, ang, w0, ws, bs)


def _upd_body(x_ref, agg_ref, deg_ref, w_ref, b_ref, o_ref):
    agg = agg_ref[0] + agg_ref[1]
    deg = deg_ref[0, :, 0:1] + deg_ref[1, :, 0:1]
    agg = agg / jnp.maximum(deg, 1.0)
    h = jnp.concatenate([x_ref[...], agg], axis=1).astype(jnp.bfloat16)
    o = jnp.dot(h, w_ref[...], preferred_element_type=jnp.float32)
    o = o + b_ref[...]
    o_ref[...] = jnp.maximum(o, 0.0)


def _upd_mlp(x, agg_p, deg_p, w, b):
    grid = NN // R_BLK
    return pl.pallas_call(
        _upd_body,
        grid=(grid,),
        in_specs=[
            pl.BlockSpec((R_BLK, D), lambda i: (i, 0)),
            pl.BlockSpec((NC, R_BLK, D), lambda i: (0, i, 0)),
            pl.BlockSpec((NC, R_BLK, D), lambda i: (0, i, 0)),
            pl.BlockSpec((2 * D, D), lambda i: (0, 0)),
            pl.BlockSpec((1, D), lambda i: (0, 0)),
        ],
        out_specs=pl.BlockSpec((R_BLK, D), lambda i: (i, 0)),
        out_shape=jax.ShapeDtypeStruct((NN, D), jnp.float32),
    )(x, agg_p, deg_p, w, b)


def _edge_stack(gs_ref, gd_ref, w0_ref, ws_ref, bs_ref):
    hf = jnp.concatenate([gs_ref[...], gd_ref[...]], axis=1)
    hr = jnp.concatenate([gd_ref[...], gs_ref[...]], axis=1)
    h = jnp.concatenate([hf, hr], axis=0)
    a = jnp.dot(h, w0_ref[...], preferred_element_type=jnp.float32)
    a = jnp.maximum(a + bs_ref[0], 0.0).astype(jnp.bfloat16)
    a = _chain(a, ws_ref, bs_ref, 5)
    return a[:E_BLK], a[E_BLK:]


def _edge1_body(gs_ref, gd_ref, w0_ref, ws_ref, bs_ref, ef_ref, sl_ref):
    e1, e2 = _edge_stack(gs_ref, gd_ref, w0_ref, ws_ref, bs_ref)

    @pl.when(pl.program_id(0) == 0)
    def _():
        sl_ref[...] = jnp.zeros((1, 1), jnp.float32)

    d = e1 - e2
    sl_ref[...] += jnp.sum(d * d).reshape(1, 1)
    ef_ref[...] = (0.5 * (e1 + e2)).astype(jnp.bfloat16)


def _edge1_mlp(gs, gd, w0, ws, bs):
    grid = NE // E_BLK
    return pl.pallas_call(
        _edge1_body,
        grid=(grid,),
        in_specs=[
            pl.BlockSpec((E_BLK, D), lambda i: (i, 0)),
            pl.BlockSpec((E_BLK, D), lambda i: (i, 0)),
            pl.BlockSpec((2 * D, DH), lambda i: (0, 0)),
            pl.BlockSpec((5, DH, DH), lambda i: (0, 0, 0)),
            pl.BlockSpec((6, DH), lambda i: (0, 0)),
        ],
        out_specs=[
            pl.BlockSpec((E_BLK, DH), lambda i: (i, 0)),
            pl.BlockSpec((1, 1), lambda i: (0, 0)),
        ],
        out_shape=[
            jax.ShapeDtypeStruct((NE, DH), jnp.bfloat16),
            jax.ShapeDtypeStruct((1, 1), jnp.float32),
        ],
    )(gs, gd, w0, ws, bs)


def _edge2_body(gs_ref, gd_ref, ef1_ref, w0_ref, ws_ref, bs_ref,
                wfe_ref, wfp_ref, fb_ref, ef_ref, sl_ref):
    e1, e2 = _edge_stack(gs_ref, gd_ref, w0_ref, ws_ref, bs_ref)

    @pl.when(pl.program_id(0) == 0)
    def _():
        sl_ref[...] = jnp.zeros((1, 1), jnp.float32)

    d = e1 - e2
    sl_ref[...] += jnp.sum(d * d).reshape(1, 1)
    e = (0.5 * (e1 + e2)).astype(jnp.bfloat16)
    o = jnp.dot(e, wfe_ref[...], preferred_element_type=jnp.float32)
    o += jnp.dot(ef1_ref[...], wfp_ref[...], preferred_element_type=jnp.float32)
    ef_ref[...] = o + fb_ref[...]


def _edge2_mlp(gs, gd, ef1, w0, ws, bs, wfe, wfp, fb):
    grid = NE // E_BLK
    return pl.pallas_call(
        _edge2_body,
        grid=(grid,),
        in_specs=[
            pl.BlockSpec((E_BLK, D), lambda i: (i, 0)),
            pl.BlockSpec((E_BLK, D), lambda i: (i, 0)),
            pl.BlockSpec((E_BLK, DH), lambda i: (i, 0)),
            pl.BlockSpec((2 * D, DH), lambda i: (0, 0)),
            pl.BlockSpec((5, DH, DH), lambda i: (0, 0, 0)),
            pl.BlockSpec((6, DH), lambda i: (0, 0)),
            pl.BlockSpec((DH, D), lambda i: (0, 0)),
            pl.BlockSpec((DH, D), lambda i: (0, 0)),
            pl.BlockSpec((1, D), lambda i: (0, 0)),
        ],
        out_specs=[
            pl.BlockSpec((E_BLK, D), lambda i: (i, 0)),
            pl.BlockSpec((1, 1), lambda i: (0, 0)),
        ],
        out_shape=[
            jax.ShapeDtypeStruct((NE, D), jnp.float32),
            jax.ShapeDtypeStruct((1, 1), jnp.float32),
        ],
    )(gs, gd, ef1, w0, ws, bs, wfe, wfp, fb)


def _stack_mlp(p):
    w0 = p[0][0].astype(jnp.bfloat16)
    ws = jnp.stack([w for (w, _) in p[1:]]).astype(jnp.bfloat16)
    bs = jnp.stack([b for (_, b) in p])
    return w0, ws, bs


def kernel(node_features, edge_index, angles, gt_edges, params):
    del gt_edges
    src = edge_index[0]
    dst = edge_index[1]
    idx_all = jnp.concatenate([src, dst])
    ang = angles.reshape(NE, 1)

    m1w0, m1ws, m1bs = _stack_mlp(params["nc1"]["msg"])
    m2w0, m2ws, m2bs = _stack_mlp(params["nc2"]["msg"])
    e1w0, e1ws, e1bs = _stack_mlp(params["ec1"]["edge"])
    e2w0, e2ws, e2bs = _stack_mlp(params["ec2"]["edge"])
    u1w, u1b = params["nc1"]["upd"][0]
    u2w, u2b = params["nc2"]["upd"][0]
    fw, fb = params["ec2"]["fuse"][0]
    u1w = u1w.astype(jnp.bfloat16)
    u2w = u2w.astype(jnp.bfloat16)
    fw = fw.astype(jnp.bfloat16)
    u1b = u1b.reshape(1, D)
    u2b = u2b.reshape(1, D)
    fb = fb.reshape(1, D)
    wfe, wfp = fw[:DH], fw[DH:]

    zeros_d = jnp.zeros((NN, D), jnp.float32)
    ones_d = jnp.ones((CH, D), jnp.float32)

    x0 = node_features

    dp = _segment_count(dst, zeros_d, ones_d)

    g0 = _gather_rows(x0, idx_all)
    m1 = _msg_mlp(g0[:NE], g0[NE:], ang, m1w0, m1ws, m1bs)
    a1 = _segment_sum(m1, dst, zeros_d)
    x1 = _upd_mlp(x0, a1, dp, u1w, u1b)

    g1 = _gather_rows(x1, idx_all)
    g1s, g1d = g1[:NE], g1[NE:]

    m2 = _msg_mlp(g1s, g1d, ang, m2w0, m2ws, m2bs)
    a2 = _segment_sum(m2, dst, zeros_d)
    x2 = _upd_mlp(x1, a2, dp, u2w, u2b)
    g2 = _gather_rows(x2, idx_all)

    ef1, sl1 = _edge1_mlp(g1s, g1d, e1w0, e1ws, e1bs)

    ef, sl2 = _edge2_mlp(g2[:NE], g2[NE:], ef1, e2w0, e2ws, e2bs, wfe, wfp, fb)

    side_loss = (sl1[0, 0] + sl2[0, 0]) / (2.0 * NE * DH)
    return ef, side_loss


# speedup vs baseline: 1.1024x; 1.1009x over previous
import functools

import jax
import jax.numpy as jnp
from jax import lax
from jax.experimental import pallas as pl
from jax.experimental.pallas import tpu as pltpu
from jax.experimental.pallas import tpu_sc as plsc

NN = 10000
NE = 160000
D = 128
DH = 384

NC = 2
NS = 16
NW = NC * NS
CH = 128

E_BLK = 2000
R_BLK = 2000

_SC_MESH = plsc.VectorSubcoreMesh(core_axis_name="c", subcore_axis_name="s")


def _gather_rows(table, idx):
    B = idx.shape[0]
    n_chunks = B // CH
    max_nj = -(-n_chunks // NW)
    dt = table.dtype
    W = table.shape[1]

    @functools.partial(
        pl.kernel,
        out_type=jax.ShapeDtypeStruct((B, W), dt),
        mesh=_SC_MESH,
        scratch_types=[
            pltpu.VMEM((CH,), jnp.int32),
            pltpu.VMEM((CH, W), dt),
            pltpu.SemaphoreType.DMA,
        ],
    )
    def k(table_hbm, idx_hbm, out_hbm, idx_v, rows_v, sem):
        wid = lax.axis_index("s") * NC + lax.axis_index("c")

        @pl.loop(0, max_nj)
        def _(j):
            c = j * NW + wid

            @pl.when(c < n_chunks)
            def _():
                off = c * CH
                pltpu.sync_copy(idx_hbm.at[pl.ds(off, CH)], idx_v)
                pltpu.async_copy(table_hbm.at[idx_v], rows_v, sem).wait()
                pltpu.sync_copy(rows_v, out_hbm.at[pl.ds(off, CH)])

    return k(table, idx)


def _per_sub_slices(sid, fn):
    rps = 624

    @pl.when(sid < NS - 1)
    def _():
        fn(sid * rps, rps)

    @pl.when(sid == NS - 1)
    def _():
        fn((NS - 1) * rps, NN - (NS - 1) * rps)


def _segment_sum(m, dst, zeros_d):
    n_chunks = NE // CH
    max_nj = -(-n_chunks // NW)

    @functools.partial(
        pl.kernel,
        out_type=jax.ShapeDtypeStruct((NC, NN, D), jnp.float32),
        mesh=_SC_MESH,
        scratch_types=[
            pltpu.VMEM((CH,), jnp.int32),
            pltpu.VMEM((CH, D), jnp.float32),
            pltpu.VMEM_SHARED((NN, D), jnp.float32),
            pltpu.SemaphoreType.DMA,
        ],
    )
    def k(m_hbm, dst_hbm, z_d_hbm, agg_hbm, idx_v, rows_v, acc_sh, sem):
        cid = lax.axis_index("c")
        sid = lax.axis_index("s")
        wid = sid * NC + cid

        _per_sub_slices(sid, lambda o, n: pltpu.sync_copy(
            z_d_hbm.at[pl.ds(o, n)], acc_sh.at[pl.ds(o, n)]))
        plsc.subcore_barrier()

        @pl.loop(0, max_nj)
        def _(j):
            c = j * NW + wid

            @pl.when(c < n_chunks)
            def _():
                off = c * CH
                pltpu.sync_copy(dst_hbm.at[pl.ds(off, CH)], idx_v)
                pltpu.sync_copy(m_hbm.at[pl.ds(off, CH)], rows_v)
                pltpu.sync_copy(rows_v, acc_sh.at[idx_v], add=True)

        plsc.subcore_barrier()
        _per_sub_slices(sid, lambda o, n: pltpu.sync_copy(
            acc_sh.at[pl.ds(o, n)], agg_hbm.at[cid, pl.ds(o, n)]))

    return k(m, dst, zeros_d)


def _segment_count(dst, zeros_d, ones_d):
    n_chunks = NE // CH
    max_nj = -(-n_chunks // NW)

    @functools.partial(
        pl.kernel,
        out_type=jax.ShapeDtypeStruct((NC, NN, D), jnp.float32),
        mesh=_SC_MESH,
        scratch_types=[
            pltpu.VMEM((CH,), jnp.int32),
            pltpu.VMEM((CH, D), jnp.float32),
            pltpu.VMEM_SHARED((NN, D), jnp.float32),
            pltpu.SemaphoreType.DMA,
        ],
    )
    def k(dst_hbm, z_d_hbm, ones_hbm, deg_hbm, idx_v, ones_v, acc_sh, sem):
        cid = lax.axis_index("c")
        sid = lax.axis_index("s")
        wid = sid * NC + cid

        _per_sub_slices(sid, lambda o, n: pltpu.sync_copy(
            z_d_hbm.at[pl.ds(o, n)], acc_sh.at[pl.ds(o, n)]))
        pltpu.sync_copy(ones_hbm, ones_v)
        plsc.subcore_barrier()

        @pl.loop(0, max_nj)
        def _(j):
            c = j * NW + wid

            @pl.when(c < n_chunks)
            def _():
                pltpu.sync_copy(dst_hbm.at[pl.ds(c * CH, CH)], idx_v)
                pltpu.sync_copy(ones_v, acc_sh.at[idx_v], add=True)

        plsc.subcore_barrier()
        _per_sub_slices(sid, lambda o, n: pltpu.sync_copy(
            acc_sh.at[pl.ds(o, n)], deg_hbm.at[cid, pl.ds(o, n)]))

    return k(dst, zeros_d, ones_d)


def _chain(a, ws_ref, bs_ref, bs_f32, n_hidden):
    for i in range(n_hidden - 1):
        a = jnp.dot(a, ws_ref[i], preferred_element_type=jnp.float32)
        a = jnp.maximum(a.astype(jnp.bfloat16) + bs_ref[i + 1],
                        jnp.bfloat16(0.0))
    a = jnp.dot(a, ws_ref[n_hidden - 1], preferred_element_type=jnp.float32)
    return a + bs_f32


def _msg_body(gs_ref, gd_ref, ang_ref, w0_ref, ws_ref, bs_ref, bl_ref, m_ref):
    h = jnp.concatenate([gs_ref[...], gd_ref[...]], axis=1)
    a = jnp.dot(h, w0_ref[...], preferred_element_type=jnp.float32)
    a = jnp.maximum(a.astype(jnp.bfloat16) + bs_ref[0], jnp.bfloat16(0.0))
    a = _chain(a, ws_ref, bs_ref, bl_ref[...], 5)
    m_ref[...] = a * ang_ref[...]


N_BLKS = NE // E_BLK


def _msg_mlp(g, ang, w0, ws, bs, bl):
    return pl.pallas_call(
        _msg_body,
        grid=(N_BLKS,),
        in_specs=[
            pl.BlockSpec((E_BLK, D), lambda i: (i, 0)),
            pl.BlockSpec((E_BLK, D), lambda i: (i + N_BLKS, 0)),
            pl.BlockSpec((E_BLK, 1), lambda i: (i, 0)),
            pl.BlockSpec((2 * D, D), lambda i: (0, 0)),
            pl.BlockSpec((5, D, D), lambda i: (0, 0, 0)),
            pl.BlockSpec((5, D), lambda i: (0, 0)),
            pl.BlockSpec((1, D), lambda i: (0, 0)),
        ],
        out_specs=pl.BlockSpec((E_BLK, D), lambda i: (i, 0)),
        out_shape=jax.ShapeDtypeStruct((NE, D), jnp.float32),
    )(g, g, ang, w0, ws, bs, bl)


def _upd_body(x_ref, agg_ref, deg_ref, w_ref, b_ref, o_ref):
    agg = agg_ref[0] + agg_ref[1]
    deg = deg_ref[0, :, 0:1] + deg_ref[1, :, 0:1]
    agg = agg / jnp.maximum(deg, 1.0)
    h = jnp.concatenate([x_ref[...], agg], axis=1).astype(jnp.bfloat16)
    o = jnp.dot(h, w_ref[...], preferred_element_type=jnp.float32)
    o = o + b_ref[...]
    o_ref[...] = jnp.maximum(o, 0.0)


def _upd_mlp(x, agg_p, deg_p, w, b):
    grid = NN // R_BLK
    return pl.pallas_call(
        _upd_body,
        grid=(grid,),
        in_specs=[
            pl.BlockSpec((R_BLK, D), lambda i: (i, 0)),
            pl.BlockSpec((NC, R_BLK, D), lambda i: (0, i, 0)),
            pl.BlockSpec((NC, R_BLK, D), lambda i: (0, i, 0)),
            pl.BlockSpec((2 * D, D), lambda i: (0, 0)),
            pl.BlockSpec((1, D), lambda i: (0, 0)),
        ],
        out_specs=pl.BlockSpec((R_BLK, D), lambda i: (i, 0)),
        out_shape=jax.ShapeDtypeStruct((NN, D), jnp.float32),
    )(x, agg_p, deg_p, w, b)


def _edge_stack(gs_ref, gd_ref, w0_ref, ws_ref, bs_ref, bl_ref):
    hf = jnp.concatenate([gs_ref[...], gd_ref[...]], axis=1)
    hr = jnp.concatenate([gd_ref[...], gs_ref[...]], axis=1)
    h = jnp.concatenate([hf, hr], axis=0)
    a = jnp.dot(h, w0_ref[...], preferred_element_type=jnp.float32)
    a = jnp.maximum(a.astype(jnp.bfloat16) + bs_ref[0], jnp.bfloat16(0.0))
    a = _chain(a, ws_ref, bs_ref, bl_ref[...], 5)
    return a[:E_BLK], a[E_BLK:]


def _edge1_body(gs_ref, gd_ref, w0_ref, ws_ref, bs_ref, bl_ref, ef_ref, sl_ref):
    e1, e2 = _edge_stack(gs_ref, gd_ref, w0_ref, ws_ref, bs_ref, bl_ref)

    @pl.when(pl.program_id(0) == 0)
    def _():
        sl_ref[...] = jnp.zeros((1, 1), jnp.float32)

    d = e1 - e2
    sl_ref[...] += jnp.sum(d * d).reshape(1, 1)
    ef_ref[...] = (0.5 * (e1 + e2)).astype(jnp.bfloat16)


def _edge1_mlp(g, w0, ws, bs, bl):
    return pl.pallas_call(
        _edge1_body,
        grid=(N_BLKS,),
        in_specs=[
            pl.BlockSpec((E_BLK, D), lambda i: (i, 0)),
            pl.BlockSpec((E_BLK, D), lambda i: (i + N_BLKS, 0)),
            pl.BlockSpec((2 * D, DH), lambda i: (0, 0)),
            pl.BlockSpec((5, DH, DH), lambda i: (0, 0, 0)),
            pl.BlockSpec((5, DH), lambda i: (0, 0)),
            pl.BlockSpec((1, DH), lambda i: (0, 0)),
        ],
        out_specs=[
            pl.BlockSpec((E_BLK, DH), lambda i: (i, 0)),
            pl.BlockSpec((1, 1), lambda i: (0, 0)),
        ],
        out_shape=[
            jax.ShapeDtypeStruct((NE, DH), jnp.bfloat16),
            jax.ShapeDtypeStruct((1, 1), jnp.float32),
        ],
    )(g, g, w0, ws, bs, bl)


def _edge2_body(gs_ref, gd_ref, ef1_ref, w0_ref, ws_ref, bs_ref, bl_ref,
                wfe_ref, wfp_ref, fb_ref, ef_ref, sl_ref):
    e1, e2 = _edge_stack(gs_ref, gd_ref, w0_ref, ws_ref, bs_ref, bl_ref)

    @pl.when(pl.program_id(0) == 0)
    def _():
        sl_ref[...] = jnp.zeros((1, 1), jnp.float32)

    d = e1 - e2
    sl_ref[...] += jnp.sum(d * d).reshape(1, 1)
    e = (0.5 * (e1 + e2)).astype(jnp.bfloat16)
    o = jnp.dot(e, wfe_ref[...], preferred_element_type=jnp.float32)
    o += jnp.dot(ef1_ref[...], wfp_ref[...], preferred_element_type=jnp.float32)
    ef_ref[...] = o + fb_ref[...]


def _edge2_mlp(g, ef1, w0, ws, bs, bl, wfe, wfp, fb):
    return pl.pallas_call(
        _edge2_body,
        grid=(N_BLKS,),
        in_specs=[
            pl.BlockSpec((E_BLK, D), lambda i: (i, 0)),
            pl.BlockSpec((E_BLK, D), lambda i: (i + N_BLKS, 0)),
            pl.BlockSpec((E_BLK, DH), lambda i: (i, 0)),
            pl.BlockSpec((2 * D, DH), lambda i: (0, 0)),
            pl.BlockSpec((5, DH, DH), lambda i: (0, 0, 0)),
            pl.BlockSpec((5, DH), lambda i: (0, 0)),
            pl.BlockSpec((1, DH), lambda i: (0, 0)),
            pl.BlockSpec((DH, D), lambda i: (0, 0)),
            pl.BlockSpec((DH, D), lambda i: (0, 0)),
            pl.BlockSpec((1, D), lambda i: (0, 0)),
        ],
        out_specs=[
            pl.BlockSpec((E_BLK, D), lambda i: (i, 0)),
            pl.BlockSpec((1, 1), lambda i: (0, 0)),
        ],
        out_shape=[
            jax.ShapeDtypeStruct((NE, D), jnp.float32),
            jax.ShapeDtypeStruct((1, 1), jnp.float32),
        ],
    )(g, g, ef1, w0, ws, bs, bl, wfe, wfp, fb)


def _stack_mlp(p):
    w0 = p[0][0].astype(jnp.bfloat16)
    ws = jnp.stack([w for (w, _) in p[1:]]).astype(jnp.bfloat16)
    bs = jnp.stack([b for (_, b) in p[:-1]]).astype(jnp.bfloat16)
    bl = p[-1][1].reshape(1, -1)
    return w0, ws, bs, bl


def kernel(node_features, edge_index, angles, gt_edges, params):
    del gt_edges
    src = edge_index[0]
    dst = edge_index[1]
    idx_all = jnp.concatenate([src, dst])
    ang = angles.reshape(NE, 1)

    m1w0, m1ws, m1bs, m1bl = _stack_mlp(params["nc1"]["msg"])
    m2w0, m2ws, m2bs, m2bl = _stack_mlp(params["nc2"]["msg"])
    e1w0, e1ws, e1bs, e1bl = _stack_mlp(params["ec1"]["edge"])
    e2w0, e2ws, e2bs, e2bl = _stack_mlp(params["ec2"]["edge"])
    u1w, u1b = params["nc1"]["upd"][0]
    u2w, u2b = params["nc2"]["upd"][0]
    fw, fb = params["ec2"]["fuse"][0]
    u1w = u1w.astype(jnp.bfloat16)
    u2w = u2w.astype(jnp.bfloat16)
    fw = fw.astype(jnp.bfloat16)
    u1b = u1b.reshape(1, D)
    u2b = u2b.reshape(1, D)
    fb = fb.reshape(1, D)
    wfe, wfp = fw[:DH], fw[DH:]

    zeros_d = jnp.zeros((NN, D), jnp.float32)
    ones_d = jnp.ones((CH, D), jnp.float32)

    x0 = node_features

    dp = _segment_count(dst, zeros_d, ones_d)

    g0 = _gather_rows(x0, idx_all)
    m1 = _msg_mlp(g0, ang, m1w0, m1ws, m1bs, m1bl)
    a1 = _segment_sum(m1, dst, zeros_d)
    x1 = _upd_mlp(x0, a1, dp, u1w, u1b)

    g1 = _gather_rows(x1, idx_all)

    m2 = _msg_mlp(g1, ang, m2w0, m2ws, m2bs, m2bl)
    a2 = _segment_sum(m2, dst, zeros_d)
    x2 = _upd_mlp(x1, a2, dp, u2w, u2b)
    g2 = _gather_rows(x2, idx_all)

    ef1, sl1 = _edge1_mlp(g1, e1w0, e1ws, e1bs, e1bl)

    ef, sl2 = _edge2_mlp(g2, ef1, e2w0, e2ws, e2bs, e2bl, wfe, wfp, fb)

    side_loss = (sl1[0, 0] + sl2[0, 0]) / (2.0 * NE * DH)
    return ef, side_loss


# speedup vs baseline: 1.1524x; 1.0453x over previous
import functools

import jax
import jax.numpy as jnp
from jax import lax
from jax.experimental import pallas as pl
from jax.experimental.pallas import tpu as pltpu
from jax.experimental.pallas import tpu_sc as plsc

NN = 10000
NE = 160000
D = 128
DH = 384

NC = 2
NS = 16
NW = NC * NS
CH = 128

E_BLK = 2000
R_BLK = 2000

_SC_MESH = plsc.VectorSubcoreMesh(core_axis_name="c", subcore_axis_name="s")


def _gather_rows(table, idx):
    B = idx.shape[0]
    n_chunks = B // CH
    npw = -(-n_chunks // NW)
    last_n = n_chunks - (NW - 1) * npw

    @functools.partial(
        pl.kernel,
        out_type=jax.ShapeDtypeStruct((B, D), jnp.float32),
        mesh=_SC_MESH,
        scratch_types=[
            pltpu.VMEM((npw * CH,), jnp.int32),
            pltpu.VMEM((CH, D), jnp.float32),
            pltpu.VMEM((CH, D), jnp.float32),
            pltpu.SemaphoreType.DMA,
            pltpu.SemaphoreType.DMA,
            pltpu.SemaphoreType.DMA,
        ],
    )
    def k(table_hbm, idx_hbm, out_hbm, idx_v, r0_v, r1_v, gsem, w0sem, w1sem):
        wid = lax.axis_index("s") * NC + lax.axis_index("c")
        base = wid * npw
        my_n = jnp.where(wid < NW - 1, npw, last_n)
        rows = (r0_v, r1_v)
        wsems = (w0sem, w1sem)

        @pl.when(wid < NW - 1)
        def _():
            pltpu.sync_copy(idx_hbm.at[pl.ds(base * CH, npw * CH)], idx_v)

        @pl.when(wid == NW - 1)
        def _():
            pltpu.sync_copy(idx_hbm.at[pl.ds(base * CH, last_n * CH)],
                            idx_v.at[pl.ds(0, last_n * CH)])

        @pl.loop(0, npw, step=2)
        def _(t0):
            for b in range(2):
                t = t0 + b

                @pl.when((t >= 2) & (t < my_n))
                def _():
                    pltpu.make_async_copy(
                        rows[b], out_hbm.at[pl.ds(0, CH)], wsems[b]).wait()

            for b in range(2):
                t = t0 + b

                @pl.when(t < my_n)
                def _():
                    pltpu.async_copy(
                        table_hbm.at[idx_v.at[pl.ds(t * CH, CH)]],
                        rows[b], gsem)

            for b in range(2):
                t = t0 + b

                @pl.when(t < my_n)
                def _():
                    pltpu.make_async_copy(
                        table_hbm.at[idx_v.at[pl.ds(t * CH, CH)]],
                        rows[b], gsem).wait()

            for b in range(2):
                t = t0 + b

                @pl.when(t < my_n)
                def _():
                    pltpu.async_copy(
                        rows[b], out_hbm.at[pl.ds((base + t) * CH, CH)],
                        wsems[b])

        @pl.when(my_n >= 2)
        def _():
            pltpu.make_async_copy(r0_v, out_hbm.at[pl.ds(0, CH)], w0sem).wait()
            pltpu.make_async_copy(r1_v, out_hbm.at[pl.ds(0, CH)], w1sem).wait()

        @pl.when(my_n == 1)
        def _():
            pltpu.make_async_copy(r0_v, out_hbm.at[pl.ds(0, CH)], w0sem).wait()

    return k(table, idx)


def _per_sub_slices(sid, fn):
    rps = 624

    @pl.when(sid < NS - 1)
    def _():
        fn(sid * rps, rps)

    @pl.when(sid == NS - 1)
    def _():
        fn((NS - 1) * rps, NN - (NS - 1) * rps)


def _segment_sum(m, dst, zeros_d):
    n_chunks = NE // CH
    max_nj = -(-n_chunks // NW)

    @functools.partial(
        pl.kernel,
        out_type=jax.ShapeDtypeStruct((NC, NN, D), jnp.float32),
        mesh=_SC_MESH,
        scratch_types=[
            pltpu.VMEM((CH,), jnp.int32),
            pltpu.VMEM((CH, D), jnp.float32),
            pltpu.VMEM_SHARED((NN, D), jnp.float32),
            pltpu.SemaphoreType.DMA,
        ],
    )
    def k(m_hbm, dst_hbm, z_d_hbm, agg_hbm, idx_v, rows_v, acc_sh, sem):
        cid = lax.axis_index("c")
        sid = lax.axis_index("s")
        wid = sid * NC + cid

        _per_sub_slices(sid, lambda o, n: pltpu.sync_copy(
            z_d_hbm.at[pl.ds(o, n)], acc_sh.at[pl.ds(o, n)]))
        plsc.subcore_barrier()

        @pl.loop(0, max_nj)
        def _(j):
            c = j * NW + wid

            @pl.when(c < n_chunks)
            def _():
                off = c * CH
                pltpu.sync_copy(dst_hbm.at[pl.ds(off, CH)], idx_v)
                pltpu.sync_copy(m_hbm.at[pl.ds(off, CH)], rows_v)
                pltpu.sync_copy(rows_v, acc_sh.at[idx_v], add=True)

        plsc.subcore_barrier()
        _per_sub_slices(sid, lambda o, n: pltpu.sync_copy(
            acc_sh.at[pl.ds(o, n)], agg_hbm.at[cid, pl.ds(o, n)]))

    return k(m, dst, zeros_d)


def _segment_count(dst, zeros_d, ones_d):
    n_chunks = NE // CH
    max_nj = -(-n_chunks // NW)

    @functools.partial(
        pl.kernel,
        out_type=jax.ShapeDtypeStruct((NC, NN, D), jnp.float32),
        mesh=_SC_MESH,
        scratch_types=[
            pltpu.VMEM((CH,), jnp.int32),
            pltpu.VMEM((CH, D), jnp.float32),
            pltpu.VMEM_SHARED((NN, D), jnp.float32),
            pltpu.SemaphoreType.DMA,
        ],
    )
    def k(dst_hbm, z_d_hbm, ones_hbm, deg_hbm, idx_v, ones_v, acc_sh, sem):
        cid = lax.axis_index("c")
        sid = lax.axis_index("s")
        wid = sid * NC + cid

        _per_sub_slices(sid, lambda o, n: pltpu.sync_copy(
            z_d_hbm.at[pl.ds(o, n)], acc_sh.at[pl.ds(o, n)]))
        pltpu.sync_copy(ones_hbm, ones_v)
        plsc.subcore_barrier()

        @pl.loop(0, max_nj)
        def _(j):
            c = j * NW + wid

            @pl.when(c < n_chunks)
            def _():
                pltpu.sync_copy(dst_hbm.at[pl.ds(c * CH, CH)], idx_v)
                pltpu.sync_copy(ones_v, acc_sh.at[idx_v], add=True)

        plsc.subcore_barrier()
        _per_sub_slices(sid, lambda o, n: pltpu.sync_copy(
            acc_sh.at[pl.ds(o, n)], deg_hbm.at[cid, pl.ds(o, n)]))

    return k(dst, zeros_d, ones_d)


def _chain(a, ws_ref, bs_ref, bs_f32, n_hidden):
    for i in range(n_hidden - 1):
        a = jnp.dot(a, ws_ref[i], preferred_element_type=jnp.float32)
        a = jnp.maximum(a.astype(jnp.bfloat16) + bs_ref[i + 1],
                        jnp.bfloat16(0.0))
    a = jnp.dot(a, ws_ref[n_hidden - 1], preferred_element_type=jnp.float32)
    return a + bs_f32


def _msg_body(gs_ref, gd_ref, ang_ref, w0_ref, ws_ref, bs_ref, bl_ref, m_ref):
    h = jnp.concatenate([gs_ref[...], gd_ref[...]], axis=1)
    a = jnp.dot(h, w0_ref[...], preferred_element_type=jnp.float32)
    a = jnp.maximum(a.astype(jnp.bfloat16) + bs_ref[0], jnp.bfloat16(0.0))
    a = _chain(a, ws_ref, bs_ref, bl_ref[...], 5)
    m_ref[...] = a * ang_ref[...]


N_BLKS = NE // E_BLK


def _msg_mlp(g, ang, w0, ws, bs, bl):
    return pl.pallas_call(
        _msg_body,
        grid=(N_BLKS,),
        in_specs=[
            pl.BlockSpec((E_BLK, D), lambda i: (i, 0)),
            pl.BlockSpec((E_BLK, D), lambda i: (i + N_BLKS, 0)),
            pl.BlockSpec((E_BLK, 1), lambda i: (i, 0)),
            pl.BlockSpec((2 * D, D), lambda i: (0, 0)),
            pl.BlockSpec((5, D, D), lambda i: (0, 0, 0)),
            pl.BlockSpec((5, D), lambda i: (0, 0)),
            pl.BlockSpec((1, D), lambda i: (0, 0)),
        ],
        out_specs=pl.BlockSpec((E_BLK, D), lambda i: (i, 0)),
        out_shape=jax.ShapeDtypeStruct((NE, D), jnp.float32),
    )(g, g, ang, w0, ws, bs, bl)


def _upd_body(x_ref, agg_ref, deg_ref, w_ref, b_ref, o_ref):
    agg = agg_ref[0] + agg_ref[1]
    deg = deg_ref[0, :, 0:1] + deg_ref[1, :, 0:1]
    agg = agg / jnp.maximum(deg, 1.0)
    h = jnp.concatenate([x_ref[...], agg], axis=1).astype(jnp.bfloat16)
    o = jnp.dot(h, w_ref[...], preferred_element_type=jnp.float32)
    o = o + b_ref[...]
    o_ref[...] = jnp.maximum(o, 0.0)


def _upd_mlp(x, agg_p, deg_p, w, b):
    grid = NN // R_BLK
    return pl.pallas_call(
        _upd_body,
        grid=(grid,),
        in_specs=[
            pl.BlockSpec((R_BLK, D), lambda i: (i, 0)),
            pl.BlockSpec((NC, R_BLK, D), lambda i: (0, i, 0)),
            pl.BlockSpec((NC, R_BLK, D), lambda i: (0, i, 0)),
            pl.BlockSpec((2 * D, D), lambda i: (0, 0)),
            pl.BlockSpec((1, D), lambda i: (0, 0)),
        ],
        out_specs=pl.BlockSpec((R_BLK, D), lambda i: (i, 0)),
        out_shape=jax.ShapeDtypeStruct((NN, D), jnp.float32),
    )(x, agg_p, deg_p, w, b)


def _edge_stack(gs_ref, gd_ref, w0_ref, ws_ref, bs_ref, bl_ref):
    hf = jnp.concatenate([gs_ref[...], gd_ref[...]], axis=1)
    hr = jnp.concatenate([gd_ref[...], gs_ref[...]], axis=1)
    h = jnp.concatenate([hf, hr], axis=0)
    a = jnp.dot(h, w0_ref[...], preferred_element_type=jnp.float32)
    a = jnp.maximum(a.astype(jnp.bfloat16) + bs_ref[0], jnp.bfloat16(0.0))
    a = _chain(a, ws_ref, bs_ref, bl_ref[...], 5)
    return a[:E_BLK], a[E_BLK:]


def _edge1_body(gs_ref, gd_ref, w0_ref, ws_ref, bs_ref, bl_ref, ef_ref, sl_ref):
    e1, e2 = _edge_stack(gs_ref, gd_ref, w0_ref, ws_ref, bs_ref, bl_ref)

    @pl.when(pl.program_id(0) == 0)
    def _():
        sl_ref[...] = jnp.zeros((1, 1), jnp.float32)

    d = e1 - e2
    sl_ref[...] += jnp.sum(d * d).reshape(1, 1)
    ef_ref[...] = (0.5 * (e1 + e2)).astype(jnp.bfloat16)


def _edge1_mlp(g, w0, ws, bs, bl):
    return pl.pallas_call(
        _edge1_body,
        grid=(N_BLKS,),
        in_specs=[
            pl.BlockSpec((E_BLK, D), lambda i: (i, 0)),
            pl.BlockSpec((E_BLK, D), lambda i: (i + N_BLKS, 0)),
            pl.BlockSpec((2 * D, DH), lambda i: (0, 0)),
            pl.BlockSpec((5, DH, DH), lambda i: (0, 0, 0)),
            pl.BlockSpec((5, DH), lambda i: (0, 0)),
            pl.BlockSpec((1, DH), lambda i: (0, 0)),
        ],
        out_specs=[
            pl.BlockSpec((E_BLK, DH), lambda i: (i, 0)),
            pl.BlockSpec((1, 1), lambda i: (0, 0)),
        ],
        out_shape=[
            jax.ShapeDtypeStruct((NE, DH), jnp.bfloat16),
            jax.ShapeDtypeStruct((1, 1), jnp.float32),
        ],
    )(g, g, w0, ws, bs, bl)


def _edge2_body(gs_ref, gd_ref, ef1_ref, w0_ref, ws_ref, bs_ref, bl_ref,
                wfe_ref, wfp_ref, fb_ref, ef_ref, sl_ref):
    e1, e2 = _edge_stack(gs_ref, gd_ref, w0_ref, ws_ref, bs_ref, bl_ref)

    @pl.when(pl.program_id(0) == 0)
    def _():
        sl_ref[...] = jnp.zeros((1, 1), jnp.float32)

    d = e1 - e2
    sl_ref[...] += jnp.sum(d * d).reshape(1, 1)
    e = (0.5 * (e1 + e2)).astype(jnp.bfloat16)
    o = jnp.dot(e, wfe_ref[...], preferred_element_type=jnp.float32)
    o += jnp.dot(ef1_ref[...], wfp_ref[...], preferred_element_type=jnp.float32)
    ef_ref[...] = o + fb_ref[...]


def _edge2_mlp(g, ef1, w0, ws, bs, bl, wfe, wfp, fb):
    return pl.pallas_call(
        _edge2_body,
        grid=(N_BLKS,),
        in_specs=[
            pl.BlockSpec((E_BLK, D), lambda i: (i, 0)),
            pl.BlockSpec((E_BLK, D), lambda i: (i + N_BLKS, 0)),
            pl.BlockSpec((E_BLK, DH), lambda i: (i, 0)),
            pl.BlockSpec((2 * D, DH), lambda i: (0, 0)),
            pl.BlockSpec((5, DH, DH), lambda i: (0, 0, 0)),
            pl.BlockSpec((5, DH), lambda i: (0, 0)),
            pl.BlockSpec((1, DH), lambda i: (0, 0)),
            pl.BlockSpec((DH, D), lambda i: (0, 0)),
            pl.BlockSpec((DH, D), lambda i: (0, 0)),
            pl.BlockSpec((1, D), lambda i: (0, 0)),
        ],
        out_specs=[
            pl.BlockSpec((E_BLK, D), lambda i: (i, 0)),
            pl.BlockSpec((1, 1), lambda i: (0, 0)),
        ],
        out_shape=[
            jax.ShapeDtypeStruct((NE, D), jnp.float32),
            jax.ShapeDtypeStruct((1, 1), jnp.float32),
        ],
    )(g, g, ef1, w0, ws, bs, bl, wfe, wfp, fb)


def _stack_mlp(p):
    w0 = p[0][0].astype(jnp.bfloat16)
    ws = jnp.stack([w for (w, _) in p[1:]]).astype(jnp.bfloat16)
    bs = jnp.stack([b for (_, b) in p[:-1]]).astype(jnp.bfloat16)
    bl = p[-1][1].reshape(1, -1)
    return w0, ws, bs, bl


def kernel(node_features, edge_index, angles, gt_edges, params):
    del gt_edges
    src = edge_index[0]
    dst = edge_index[1]
    idx_all = jnp.concatenate([src, dst])
    ang = angles.reshape(NE, 1)

    m1w0, m1ws, m1bs, m1bl = _stack_mlp(params["nc1"]["msg"])
    m2w0, m2ws, m2bs, m2bl = _stack_mlp(params["nc2"]["msg"])
    e1w0, e1ws, e1bs, e1bl = _stack_mlp(params["ec1"]["edge"])
    e2w0, e2ws, e2bs, e2bl = _stack_mlp(params["ec2"]["edge"])
    u1w, u1b = params["nc1"]["upd"][0]
    u2w, u2b = params["nc2"]["upd"][0]
    fw, fb = params["ec2"]["fuse"][0]
    u1w = u1w.astype(jnp.bfloat16)
    u2w = u2w.astype(jnp.bfloat16)
    fw = fw.astype(jnp.bfloat16)
    u1b = u1b.reshape(1, D)
    u2b = u2b.reshape(1, D)
    fb = fb.reshape(1, D)
    wfe, wfp = fw[:DH], fw[DH:]

    zeros_d = jnp.zeros((NN, D), jnp.float32)
    ones_d = jnp.ones((CH, D), jnp.float32)

    x0 = node_features

    dp = _segment_count(dst, zeros_d, ones_d)

    g0 = _gather_rows(x0, idx_all)
    m1 = _msg_mlp(g0, ang, m1w0, m1ws, m1bs, m1bl)
    a1 = _segment_sum(m1, dst, zeros_d)
    x1 = _upd_mlp(x0, a1, dp, u1w, u1b)

    g1 = _gather_rows(x1, idx_all)

    m2 = _msg_mlp(g1, ang, m2w0, m2ws, m2bs, m2bl)
    a2 = _segment_sum(m2, dst, zeros_d)
    x2 = _upd_mlp(x1, a2, dp, u2w, u2b)
    g2 = _gather_rows(x2, idx_all)

    ef1, sl1 = _edge1_mlp(g1, e1w0, e1ws, e1bs, e1bl)

    ef, sl2 = _edge2_mlp(g2, ef1, e2w0, e2ws, e2bs, e2bl, wfe, wfp, fb)

    side_loss = (sl1[0, 0] + sl2[0, 0]) / (2.0 * NE * DH)
    return ef, side_loss


# speedup vs baseline: 1.1767x; 1.0211x over previous
import functools

import jax
import jax.numpy as jnp
from jax import lax
from jax.experimental import pallas as pl
from jax.experimental.pallas import tpu as pltpu
from jax.experimental.pallas import tpu_sc as plsc

NN = 10000
NE = 160000
D = 128
DH = 384

NC = 2
NS = 16
NW = NC * NS
CH = 128

E_BLK = 2000
R_BLK = 2000

_SC_MESH = plsc.VectorSubcoreMesh(core_axis_name="c", subcore_axis_name="s")


def _gather_rows(table, idx):
    B = idx.shape[0]
    n_chunks = B // CH
    npw = -(-n_chunks // NW)
    last_n = n_chunks - (NW - 1) * npw

    @functools.partial(
        pl.kernel,
        out_type=jax.ShapeDtypeStruct((B, D), jnp.float32),
        mesh=_SC_MESH,
        scratch_types=[
            pltpu.VMEM((npw * CH,), jnp.int32),
            pltpu.VMEM((CH, D), jnp.float32),
            pltpu.VMEM((CH, D), jnp.float32),
            pltpu.SemaphoreType.DMA,
            pltpu.SemaphoreType.DMA,
            pltpu.SemaphoreType.DMA,
        ],
    )
    def k(table_hbm, idx_hbm, out_hbm, idx_v, r0_v, r1_v, gsem, w0sem, w1sem):
        wid = lax.axis_index("s") * NC + lax.axis_index("c")
        base = wid * npw
        my_n = jnp.where(wid < NW - 1, npw, last_n)
        rows = (r0_v, r1_v)
        wsems = (w0sem, w1sem)

        @pl.when(wid < NW - 1)
        def _():
            pltpu.sync_copy(idx_hbm.at[pl.ds(base * CH, npw * CH)], idx_v)

        @pl.when(wid == NW - 1)
        def _():
            pltpu.sync_copy(idx_hbm.at[pl.ds(base * CH, last_n * CH)],
                            idx_v.at[pl.ds(0, last_n * CH)])

        @pl.loop(0, npw, step=2)
        def _(t0):
            for b in range(2):
                t = t0 + b

                @pl.when((t >= 2) & (t < my_n))
                def _():
                    pltpu.make_async_copy(
                        rows[b], out_hbm.at[pl.ds(0, CH)], wsems[b]).wait()

            for b in range(2):
                t = t0 + b

                @pl.when(t < my_n)
                def _():
                    pltpu.async_copy(
                        table_hbm.at[idx_v.at[pl.ds(t * CH, CH)]],
                        rows[b], gsem)

            for b in range(2):
                t = t0 + b

                @pl.when(t < my_n)
                def _():
                    pltpu.make_async_copy(
                        table_hbm.at[idx_v.at[pl.ds(t * CH, CH)]],
                        rows[b], gsem).wait()

            for b in range(2):
                t = t0 + b

                @pl.when(t < my_n)
                def _():
                    pltpu.async_copy(
                        rows[b], out_hbm.at[pl.ds((base + t) * CH, CH)],
                        wsems[b])

        @pl.when(my_n >= 2)
        def _():
            pltpu.make_async_copy(r0_v, out_hbm.at[pl.ds(0, CH)], w0sem).wait()
            pltpu.make_async_copy(r1_v, out_hbm.at[pl.ds(0, CH)], w1sem).wait()

        @pl.when(my_n == 1)
        def _():
            pltpu.make_async_copy(r0_v, out_hbm.at[pl.ds(0, CH)], w0sem).wait()

    return k(table, idx)


def _per_sub_slices(sid, fn):
    rps = 624

    @pl.when(sid < NS - 1)
    def _():
        fn(sid * rps, rps)

    @pl.when(sid == NS - 1)
    def _():
        fn((NS - 1) * rps, NN - (NS - 1) * rps)


def _segment_sum(m, dst2, zeros_d):
    n_chunks = NE // CH
    npw = -(-n_chunks // NW)
    last_n = n_chunks - (NW - 1) * npw

    @functools.partial(
        pl.kernel,
        out_type=jax.ShapeDtypeStruct((NC, NN, D), jnp.float32),
        mesh=_SC_MESH,
        scratch_types=[
            pltpu.VMEM((npw, CH), jnp.int32),
            pltpu.VMEM((CH, D), jnp.float32),
            pltpu.VMEM((CH, D), jnp.float32),
            pltpu.VMEM_SHARED((NN, D), jnp.float32),
            pltpu.SemaphoreType.DMA,
            pltpu.SemaphoreType.DMA,
        ],
    )
    def k(m_hbm, dst_hbm, z_d_hbm, agg_hbm, idx_v, m0_v, m1_v, acc_sh,
          l0sem, l1sem):
        cid = lax.axis_index("c")
        sid = lax.axis_index("s")
        wid = sid * NC + cid
        base = wid * npw
        my_n = jnp.where(wid < NW - 1, npw, last_n)
        bufs = (m0_v, m1_v)
        sems = (l0sem, l1sem)

        _per_sub_slices(sid, lambda o, n: pltpu.sync_copy(
            z_d_hbm.at[pl.ds(o, n)], acc_sh.at[pl.ds(o, n)]))

        @pl.when(wid < NW - 1)
        def _():
            pltpu.sync_copy(dst_hbm.at[pl.ds(base, npw)], idx_v)

        last_ld = -(-last_n // 8) * 8

        @pl.when(wid == NW - 1)
        def _():
            pltpu.sync_copy(dst_hbm.at[pl.ds(base, last_ld)],
                            idx_v.at[pl.ds(0, last_ld)])

        plsc.subcore_barrier()

        @pl.loop(0, npw, step=2)
        def _(t0):
            for b in range(2):
                t = t0 + b

                @pl.when(t < my_n)
                def _():
                    pltpu.async_copy(m_hbm.at[pl.ds((base + t) * CH, CH)],
                                     bufs[b], sems[b])

            for b in range(2):
                t = t0 + b

                @pl.when(t < my_n)
                def _():
                    pltpu.make_async_copy(
                        m_hbm.at[pl.ds((base + t) * CH, CH)],
                        bufs[b], sems[b]).wait()
                    pltpu.sync_copy(bufs[b], acc_sh.at[idx_v.at[t]],
                                    add=True)

        plsc.subcore_barrier()
        _per_sub_slices(sid, lambda o, n: pltpu.sync_copy(
            acc_sh.at[pl.ds(o, n)], agg_hbm.at[cid, pl.ds(o, n)]))

    return k(m, dst2, zeros_d)


def _segment_count(dst2, zeros_d, ones_d):
    n_chunks = NE // CH
    npw = -(-n_chunks // NW)
    last_n = n_chunks - (NW - 1) * npw

    @functools.partial(
        pl.kernel,
        out_type=jax.ShapeDtypeStruct((NC, NN, D), jnp.float32),
        mesh=_SC_MESH,
        scratch_types=[
            pltpu.VMEM((npw, CH), jnp.int32),
            pltpu.VMEM((CH, D), jnp.float32),
            pltpu.VMEM_SHARED((NN, D), jnp.float32),
            pltpu.SemaphoreType.DMA,
        ],
    )
    def k(dst_hbm, z_d_hbm, ones_hbm, deg_hbm, idx_v, ones_v, acc_sh, sem):
        cid = lax.axis_index("c")
        sid = lax.axis_index("s")
        wid = sid * NC + cid
        base = wid * npw
        my_n = jnp.where(wid < NW - 1, npw, last_n)

        _per_sub_slices(sid, lambda o, n: pltpu.sync_copy(
            z_d_hbm.at[pl.ds(o, n)], acc_sh.at[pl.ds(o, n)]))
        pltpu.sync_copy(ones_hbm, ones_v)

        @pl.when(wid < NW - 1)
        def _():
            pltpu.sync_copy(dst_hbm.at[pl.ds(base, npw)], idx_v)

        last_ld = -(-last_n // 8) * 8

        @pl.when(wid == NW - 1)
        def _():
            pltpu.sync_copy(dst_hbm.at[pl.ds(base, last_ld)],
                            idx_v.at[pl.ds(0, last_ld)])

        plsc.subcore_barrier()

        @pl.loop(0, npw)
        def _(t):
            @pl.when(t < my_n)
            def _():
                pltpu.sync_copy(ones_v, acc_sh.at[idx_v.at[t]], add=True)

        plsc.subcore_barrier()
        _per_sub_slices(sid, lambda o, n: pltpu.sync_copy(
            acc_sh.at[pl.ds(o, n)], deg_hbm.at[cid, pl.ds(o, n)]))

    return k(dst2, zeros_d, ones_d)


def _chain(a, ws_ref, bs_ref, bs_f32, n_hidden):
    for i in range(n_hidden - 1):
        a = jnp.dot(a, ws_ref[i], preferred_element_type=jnp.float32)
        a = jnp.maximum(a.astype(jnp.bfloat16) + bs_ref[i + 1],
                        jnp.bfloat16(0.0))
    a = jnp.dot(a, ws_ref[n_hidden - 1], preferred_element_type=jnp.float32)
    return a + bs_f32


def _msg_body(gs_ref, gd_ref, ang_ref, w0_ref, ws_ref, bs_ref, bl_ref, m_ref):
    h = jnp.concatenate([gs_ref[...], gd_ref[...]], axis=1)
    a = jnp.dot(h, w0_ref[...], preferred_element_type=jnp.float32)
    a = jnp.maximum(a.astype(jnp.bfloat16) + bs_ref[0], jnp.bfloat16(0.0))
    a = _chain(a, ws_ref, bs_ref, bl_ref[...], 5)
    m_ref[...] = a * ang_ref[...]


N_BLKS = NE // E_BLK


def _msg_mlp(g, ang, w0, ws, bs, bl):
    return pl.pallas_call(
        _msg_body,
        grid=(N_BLKS,),
        in_specs=[
            pl.BlockSpec((E_BLK, D), lambda i: (i, 0)),
            pl.BlockSpec((E_BLK, D), lambda i: (i + N_BLKS, 0)),
            pl.BlockSpec((E_BLK, 1), lambda i: (i, 0)),
            pl.BlockSpec((2 * D, D), lambda i: (0, 0)),
            pl.BlockSpec((5, D, D), lambda i: (0, 0, 0)),
            pl.BlockSpec((5, D), lambda i: (0, 0)),
            pl.BlockSpec((1, D), lambda i: (0, 0)),
        ],
        out_specs=pl.BlockSpec((E_BLK, D), lambda i: (i, 0)),
        out_shape=jax.ShapeDtypeStruct((NE, D), jnp.float32),
    )(g, g, ang, w0, ws, bs, bl)


def _upd_body(x_ref, agg_ref, deg_ref, w_ref, b_ref, o_ref):
    agg = agg_ref[0] + agg_ref[1]
    deg = deg_ref[0, :, 0:1] + deg_ref[1, :, 0:1]
    agg = agg / jnp.maximum(deg, 1.0)
    h = jnp.concatenate([x_ref[...], agg], axis=1).astype(jnp.bfloat16)
    o = jnp.dot(h, w_ref[...], preferred_element_type=jnp.float32)
    o = o + b_ref[...]
    o_ref[...] = jnp.maximum(o, 0.0)


def _upd_mlp(x, agg_p, deg_p, w, b):
    grid = NN // R_BLK
    return pl.pallas_call(
        _upd_body,
        grid=(grid,),
        in_specs=[
            pl.BlockSpec((R_BLK, D), lambda i: (i, 0)),
            pl.BlockSpec((NC, R_BLK, D), lambda i: (0, i, 0)),
            pl.BlockSpec((NC, R_BLK, D), lambda i: (0, i, 0)),
            pl.BlockSpec((2 * D, D), lambda i: (0, 0)),
            pl.BlockSpec((1, D), lambda i: (0, 0)),
        ],
        out_specs=pl.BlockSpec((R_BLK, D), lambda i: (i, 0)),
        out_shape=jax.ShapeDtypeStruct((NN, D), jnp.float32),
    )(x, agg_p, deg_p, w, b)


def _edge_stack(gs_ref, gd_ref, w0_ref, ws_ref, bs_ref, bl_ref):
    hf = jnp.concatenate([gs_ref[...], gd_ref[...]], axis=1)
    hr = jnp.concatenate([gd_ref[...], gs_ref[...]], axis=1)
    h = jnp.concatenate([hf, hr], axis=0)
    a = jnp.dot(h, w0_ref[...], preferred_element_type=jnp.float32)
    a = jnp.maximum(a.astype(jnp.bfloat16) + bs_ref[0], jnp.bfloat16(0.0))
    a = _chain(a, ws_ref, bs_ref, bl_ref[...], 5)
    return a[:E_BLK], a[E_BLK:]


def _edge1_body(gs_ref, gd_ref, w0_ref, ws_ref, bs_ref, bl_ref, ef_ref, sl_ref):
    e1, e2 = _edge_stack(gs_ref, gd_ref, w0_ref, ws_ref, bs_ref, bl_ref)

    @pl.when(pl.program_id(0) == 0)
    def _():
        sl_ref[...] = jnp.zeros((1, 1), jnp.float32)

    d = e1 - e2
    sl_ref[...] += jnp.sum(d * d).reshape(1, 1)
    ef_ref[...] = (0.5 * (e1 + e2)).astype(jnp.bfloat16)


def _edge1_mlp(g, w0, ws, bs, bl):
    return pl.pallas_call(
        _edge1_body,
        grid=(N_BLKS,),
        in_specs=[
            pl.BlockSpec((E_BLK, D), lambda i: (i, 0)),
            pl.BlockSpec((E_BLK, D), lambda i: (i + N_BLKS, 0)),
            pl.BlockSpec((2 * D, DH), lambda i: (0, 0)),
            pl.BlockSpec((5, DH, DH), lambda i: (0, 0, 0)),
            pl.BlockSpec((5, DH), lambda i: (0, 0)),
            pl.BlockSpec((1, DH), lambda i: (0, 0)),
        ],
        out_specs=[
            pl.BlockSpec((E_BLK, DH), lambda i: (i, 0)),
            pl.BlockSpec((1, 1), lambda i: (0, 0)),
        ],
        out_shape=[
            jax.ShapeDtypeStruct((NE, DH), jnp.bfloat16),
            jax.ShapeDtypeStruct((1, 1), jnp.float32),
        ],
    )(g, g, w0, ws, bs, bl)


def _edge2_body(gs_ref, gd_ref, ef1_ref, w0_ref, ws_ref, bs_ref, bl_ref,
                wfe_ref, wfp_ref, fb_ref, ef_ref, sl_ref):
    e1, e2 = _edge_stack(gs_ref, gd_ref, w0_ref, ws_ref, bs_ref, bl_ref)

    @pl.when(pl.program_id(0) == 0)
    def _():
        sl_ref[...] = jnp.zeros((1, 1), jnp.float32)

    d = e1 - e2
    sl_ref[...] += jnp.sum(d * d).reshape(1, 1)
    e = (0.5 * (e1 + e2)).astype(jnp.bfloat16)
    o = jnp.dot(e, wfe_ref[...], preferred_element_type=jnp.float32)
    o += jnp.dot(ef1_ref[...], wfp_ref[...], preferred_element_type=jnp.float32)
    ef_ref[...] = o + fb_ref[...]


def _edge2_mlp(g, ef1, w0, ws, bs, bl, wfe, wfp, fb):
    return pl.pallas_call(
        _edge2_body,
        grid=(N_BLKS,),
        in_specs=[
            pl.BlockSpec((E_BLK, D), lambda i: (i, 0)),
            pl.BlockSpec((E_BLK, D), lambda i: (i + N_BLKS, 0)),
            pl.BlockSpec((E_BLK, DH), lambda i: (i, 0)),
            pl.BlockSpec((2 * D, DH), lambda i: (0, 0)),
            pl.BlockSpec((5, DH, DH), lambda i: (0, 0, 0)),
            pl.BlockSpec((5, DH), lambda i: (0, 0)),
            pl.BlockSpec((1, DH), lambda i: (0, 0)),
            pl.BlockSpec((DH, D), lambda i: (0, 0)),
            pl.BlockSpec((DH, D), lambda i: (0, 0)),
            pl.BlockSpec((1, D), lambda i: (0, 0)),
        ],
        out_specs=[
            pl.BlockSpec((E_BLK, D), lambda i: (i, 0)),
            pl.BlockSpec((1, 1), lambda i: (0, 0)),
        ],
        out_shape=[
            jax.ShapeDtypeStruct((NE, D), jnp.float32),
            jax.ShapeDtypeStruct((1, 1), jnp.float32),
        ],
    )(g, g, ef1, w0, ws, bs, bl, wfe, wfp, fb)


def _stack_mlp(p):
    w0 = p[0][0].astype(jnp.bfloat16)
    ws = jnp.stack([w for (w, _) in p[1:]]).astype(jnp.bfloat16)
    bs = jnp.stack([b for (_, b) in p[:-1]]).astype(jnp.bfloat16)
    bl = p[-1][1].reshape(1, -1)
    return w0, ws, bs, bl


def kernel(node_features, edge_index, angles, gt_edges, params):
    del gt_edges
    src = edge_index[0]
    dst = edge_index[1]
    idx_all = jnp.concatenate([src, dst])
    n_ch = NE // CH
    npw_s = -(-n_ch // NW)
    pad_rows = -(-(n_ch - (NW - 1) * npw_s) // 8) * 8 - (n_ch - (NW - 1) * npw_s)
    dst2 = jnp.concatenate(
        [dst, jnp.zeros((pad_rows * CH,), jnp.int32)]).reshape(-1, CH)
    ang = angles.reshape(NE, 1)

    m1w0, m1ws, m1bs, m1bl = _stack_mlp(params["nc1"]["msg"])
    m2w0, m2ws, m2bs, m2bl = _stack_mlp(params["nc2"]["msg"])
    e1w0, e1ws, e1bs, e1bl = _stack_mlp(params["ec1"]["edge"])
    e2w0, e2ws, e2bs, e2bl = _stack_mlp(params["ec2"]["edge"])
    u1w, u1b = params["nc1"]["upd"][0]
    u2w, u2b = params["nc2"]["upd"][0]
    fw, fb = params["ec2"]["fuse"][0]
    u1w = u1w.astype(jnp.bfloat16)
    u2w = u2w.astype(jnp.bfloat16)
    fw = fw.astype(jnp.bfloat16)
    u1b = u1b.reshape(1, D)
    u2b = u2b.reshape(1, D)
    fb = fb.reshape(1, D)
    wfe, wfp = fw[:DH], fw[DH:]

    zeros_d = jnp.zeros((NN, D), jnp.float32)
    ones_d = jnp.ones((CH, D), jnp.float32)

    x0 = node_features

    dp = _segment_count(dst2, zeros_d, ones_d)

    g0 = _gather_rows(x0, idx_all)
    m1 = _msg_mlp(g0, ang, m1w0, m1ws, m1bs, m1bl)
    a1 = _segment_sum(m1, dst2, zeros_d)
    x1 = _upd_mlp(x0, a1, dp, u1w, u1b)

    g1 = _gather_rows(x1, idx_all)

    m2 = _msg_mlp(g1, ang, m2w0, m2ws, m2bs, m2bl)
    a2 = _segment_sum(m2, dst2, zeros_d)
    x2 = _upd_mlp(x1, a2, dp, u2w, u2b)
    g2 = _gather_rows(x2, idx_all)

    ef1, sl1 = _edge1_mlp(g1, e1w0, e1ws, e1bs, e1bl)

    ef, sl2 = _edge2_mlp(g2, ef1, e2w0, e2ws, e2bs, e2bl, wfe, wfp, fb)

    side_loss = (sl1[0, 0] + sl2[0, 0]) / (2.0 * NE * DH)
    return ef, side_loss


# speedup vs baseline: 1.2078x; 1.0264x over previous
import functools

import jax
import jax.numpy as jnp
from jax import lax
from jax.experimental import pallas as pl
from jax.experimental.pallas import tpu as pltpu
from jax.experimental.pallas import tpu_sc as plsc

NN = 10000
NE = 160000
D = 128
DH = 384

NC = 2
NS = 16
NW = NC * NS
CH = 128

E_BLK = 2000
E_MSG = 4000
R_BLK = 2000

_SC_MESH = plsc.VectorSubcoreMesh(core_axis_name="c", subcore_axis_name="s")


def _gather_rows(table, idx):
    B = idx.shape[0]
    n_chunks = B // CH
    npw = -(-n_chunks // NW)
    last_n = n_chunks - (NW - 1) * npw

    @functools.partial(
        pl.kernel,
        out_type=jax.ShapeDtypeStruct((B, D), jnp.float32),
        mesh=_SC_MESH,
        scratch_types=[
            pltpu.VMEM((npw * CH,), jnp.int32),
            pltpu.VMEM((CH, D), jnp.float32),
            pltpu.VMEM((CH, D), jnp.float32),
            pltpu.SemaphoreType.DMA,
            pltpu.SemaphoreType.DMA,
            pltpu.SemaphoreType.DMA,
        ],
    )
    def k(table_hbm, idx_hbm, out_hbm, idx_v, r0_v, r1_v, gsem, w0sem, w1sem):
        wid = lax.axis_index("s") * NC + lax.axis_index("c")
        base = wid * npw
        my_n = jnp.where(wid < NW - 1, npw, last_n)
        rows = (r0_v, r1_v)
        wsems = (w0sem, w1sem)

        @pl.when(wid < NW - 1)
        def _():
            pltpu.sync_copy(idx_hbm.at[pl.ds(base * CH, npw * CH)], idx_v)

        @pl.when(wid == NW - 1)
        def _():
            pltpu.sync_copy(idx_hbm.at[pl.ds(base * CH, last_n * CH)],
                            idx_v.at[pl.ds(0, last_n * CH)])

        @pl.loop(0, npw, step=2)
        def _(t0):
            for b in range(2):
                t = t0 + b

                @pl.when((t >= 2) & (t < my_n))
                def _():
                    pltpu.make_async_copy(
                        rows[b], out_hbm.at[pl.ds(0, CH)], wsems[b]).wait()

            for b in range(2):
                t = t0 + b

                @pl.when(t < my_n)
                def _():
                    pltpu.async_copy(
                        table_hbm.at[idx_v.at[pl.ds(t * CH, CH)]],
                        rows[b], gsem)

            for b in range(2):
                t = t0 + b

                @pl.when(t < my_n)
                def _():
                    pltpu.make_async_copy(
                        table_hbm.at[idx_v.at[pl.ds(t * CH, CH)]],
                        rows[b], gsem).wait()

            for b in range(2):
                t = t0 + b

                @pl.when(t < my_n)
                def _():
                    pltpu.async_copy(
                        rows[b], out_hbm.at[pl.ds((base + t) * CH, CH)],
                        wsems[b])

        @pl.when(my_n >= 2)
        def _():
            pltpu.make_async_copy(r0_v, out_hbm.at[pl.ds(0, CH)], w0sem).wait()
            pltpu.make_async_copy(r1_v, out_hbm.at[pl.ds(0, CH)], w1sem).wait()

        @pl.when(my_n == 1)
        def _():
            pltpu.make_async_copy(r0_v, out_hbm.at[pl.ds(0, CH)], w0sem).wait()

    return k(table, idx)


def _per_sub_slices(sid, fn):
    rps = 624

    @pl.when(sid < NS - 1)
    def _():
        fn(sid * rps, rps)

    @pl.when(sid == NS - 1)
    def _():
        fn((NS - 1) * rps, NN - (NS - 1) * rps)


def _segment_sum(m, dst2, zeros_d):
    n_chunks = NE // CH
    npw = -(-n_chunks // NW)
    last_n = n_chunks - (NW - 1) * npw

    @functools.partial(
        pl.kernel,
        out_type=jax.ShapeDtypeStruct((NC, NN, D), jnp.float32),
        mesh=_SC_MESH,
        scratch_types=[
            pltpu.VMEM((npw, CH), jnp.int32),
            pltpu.VMEM((CH, D), jnp.float32),
            pltpu.VMEM((CH, D), jnp.float32),
            pltpu.VMEM_SHARED((NN, D), jnp.float32),
            pltpu.SemaphoreType.DMA,
            pltpu.SemaphoreType.DMA,
        ],
    )
    def k(m_hbm, dst_hbm, z_d_hbm, agg_hbm, idx_v, m0_v, m1_v, acc_sh,
          l0sem, l1sem):
        cid = lax.axis_index("c")
        sid = lax.axis_index("s")
        wid = sid * NC + cid
        base = wid * npw
        my_n = jnp.where(wid < NW - 1, npw, last_n)
        bufs = (m0_v, m1_v)
        sems = (l0sem, l1sem)

        _per_sub_slices(sid, lambda o, n: pltpu.sync_copy(
            z_d_hbm.at[pl.ds(o, n)], acc_sh.at[pl.ds(o, n)]))

        @pl.when(wid < NW - 1)
        def _():
            pltpu.sync_copy(dst_hbm.at[pl.ds(base, npw)], idx_v)

        last_ld = -(-last_n // 8) * 8

        @pl.when(wid == NW - 1)
        def _():
            pltpu.sync_copy(dst_hbm.at[pl.ds(base, last_ld)],
                            idx_v.at[pl.ds(0, last_ld)])

        plsc.subcore_barrier()

        @pl.loop(0, npw, step=2)
        def _(t0):
            for b in range(2):
                t = t0 + b

                @pl.when(t < my_n)
                def _():
                    pltpu.async_copy(m_hbm.at[pl.ds((base + t) * CH, CH)],
                                     bufs[b], sems[b])

            for b in range(2):
                t = t0 + b

                @pl.when(t < my_n)
                def _():
                    pltpu.make_async_copy(
                        m_hbm.at[pl.ds((base + t) * CH, CH)],
                        bufs[b], sems[b]).wait()
                    pltpu.sync_copy(bufs[b], acc_sh.at[idx_v.at[t]],
                                    add=True)

        plsc.subcore_barrier()
        _per_sub_slices(sid, lambda o, n: pltpu.sync_copy(
            acc_sh.at[pl.ds(o, n)], agg_hbm.at[cid, pl.ds(o, n)]))

    return k(m, dst2, zeros_d)


def _segment_count(dst2, zeros_d, ones_d):
    n_chunks = NE // CH
    npw = -(-n_chunks // NW)
    last_n = n_chunks - (NW - 1) * npw

    @functools.partial(
        pl.kernel,
        out_type=jax.ShapeDtypeStruct((NC, NN, D), jnp.float32),
        mesh=_SC_MESH,
        scratch_types=[
            pltpu.VMEM((npw, CH), jnp.int32),
            pltpu.VMEM((CH, D), jnp.float32),
            pltpu.VMEM_SHARED((NN, D), jnp.float32),
            pltpu.SemaphoreType.DMA,
        ],
    )
    def k(dst_hbm, z_d_hbm, ones_hbm, deg_hbm, idx_v, ones_v, acc_sh, sem):
        cid = lax.axis_index("c")
        sid = lax.axis_index("s")
        wid = sid * NC + cid
        base = wid * npw
        my_n = jnp.where(wid < NW - 1, npw, last_n)

        _per_sub_slices(sid, lambda o, n: pltpu.sync_copy(
            z_d_hbm.at[pl.ds(o, n)], acc_sh.at[pl.ds(o, n)]))
        pltpu.sync_copy(ones_hbm, ones_v)

        @pl.when(wid < NW - 1)
        def _():
            pltpu.sync_copy(dst_hbm.at[pl.ds(base, npw)], idx_v)

        last_ld = -(-last_n // 8) * 8

        @pl.when(wid == NW - 1)
        def _():
            pltpu.sync_copy(dst_hbm.at[pl.ds(base, last_ld)],
                            idx_v.at[pl.ds(0, last_ld)])

        plsc.subcore_barrier()

        @pl.loop(0, npw)
        def _(t):
            @pl.when(t < my_n)
            def _():
                pltpu.sync_copy(ones_v, acc_sh.at[idx_v.at[t]], add=True)

        plsc.subcore_barrier()
        _per_sub_slices(sid, lambda o, n: pltpu.sync_copy(
            acc_sh.at[pl.ds(o, n)], deg_hbm.at[cid, pl.ds(o, n)]))

    return k(dst2, zeros_d, ones_d)


def _chain(a, ws_ref, bs_ref, bs_f32, n_hidden):
    for i in range(n_hidden - 1):
        a = jnp.dot(a, ws_ref[i], preferred_element_type=jnp.float32)
        a = jnp.maximum(a.astype(jnp.bfloat16) + bs_ref[i + 1],
                        jnp.bfloat16(0.0))
    a = jnp.dot(a, ws_ref[n_hidden - 1], preferred_element_type=jnp.float32)
    return a + bs_f32


def _msg_body(gs_ref, gd_ref, ang_ref, w0_ref, ws_ref, bs_ref, bl_ref, m_ref):
    h = jnp.concatenate([gs_ref[...], gd_ref[...]], axis=1)
    a = jnp.dot(h, w0_ref[...], preferred_element_type=jnp.float32)
    a = jnp.maximum(a.astype(jnp.bfloat16) + bs_ref[0], jnp.bfloat16(0.0))
    a = _chain(a, ws_ref, bs_ref, bl_ref[...], 5)
    m_ref[...] = a * ang_ref[...]


N_BLKS = NE // E_BLK
N_BLKS_MSG = NE // E_MSG


def _msg_mlp(g, ang, w0, ws, bs, bl):
    return pl.pallas_call(
        _msg_body,
        grid=(N_BLKS_MSG,),
        in_specs=[
            pl.BlockSpec((E_MSG, D), lambda i: (i, 0)),
            pl.BlockSpec((E_MSG, D), lambda i: (i + N_BLKS_MSG, 0)),
            pl.BlockSpec((E_MSG, 1), lambda i: (i, 0)),
            pl.BlockSpec((2 * D, D), lambda i: (0, 0)),
            pl.BlockSpec((5, D, D), lambda i: (0, 0, 0)),
            pl.BlockSpec((5, D), lambda i: (0, 0)),
            pl.BlockSpec((1, D), lambda i: (0, 0)),
        ],
        out_specs=pl.BlockSpec((E_MSG, D), lambda i: (i, 0)),
        out_shape=jax.ShapeDtypeStruct((NE, D), jnp.float32),
    )(g, g, ang, w0, ws, bs, bl)


def _upd_body(x_ref, agg_ref, deg_ref, w_ref, b_ref, o_ref):
    agg = agg_ref[0] + agg_ref[1]
    deg = deg_ref[0, :, 0:1] + deg_ref[1, :, 0:1]
    agg = agg / jnp.maximum(deg, 1.0)
    h = jnp.concatenate([x_ref[...], agg], axis=1).astype(jnp.bfloat16)
    o = jnp.dot(h, w_ref[...], preferred_element_type=jnp.float32)
    o = o + b_ref[...]
    o_ref[...] = jnp.maximum(o, 0.0)


def _upd_mlp(x, agg_p, deg_p, w, b):
    grid = NN // R_BLK
    return pl.pallas_call(
        _upd_body,
        grid=(grid,),
        in_specs=[
            pl.BlockSpec((R_BLK, D), lambda i: (i, 0)),
            pl.BlockSpec((NC, R_BLK, D), lambda i: (0, i, 0)),
            pl.BlockSpec((NC, R_BLK, D), lambda i: (0, i, 0)),
            pl.BlockSpec((2 * D, D), lambda i: (0, 0)),
            pl.BlockSpec((1, D), lambda i: (0, 0)),
        ],
        out_specs=pl.BlockSpec((R_BLK, D), lambda i: (i, 0)),
        out_shape=jax.ShapeDtypeStruct((NN, D), jnp.float32),
    )(x, agg_p, deg_p, w, b)


def _edge_stack(gs_ref, gd_ref, w0_ref, ws_ref, bs_ref, bl_ref):
    hf = jnp.concatenate([gs_ref[...], gd_ref[...]], axis=1)
    hr = jnp.concatenate([gd_ref[...], gs_ref[...]], axis=1)
    h = jnp.concatenate([hf, hr], axis=0)
    a = jnp.dot(h, w0_ref[...], preferred_element_type=jnp.float32)
    a = jnp.maximum(a.astype(jnp.bfloat16) + bs_ref[0], jnp.bfloat16(0.0))
    a = _chain(a, ws_ref, bs_ref, bl_ref[...], 5)
    return a[:E_BLK], a[E_BLK:]


def _edge1_body(gs_ref, gd_ref, w0_ref, ws_ref, bs_ref, bl_ref, ef_ref, sl_ref):
    e1, e2 = _edge_stack(gs_ref, gd_ref, w0_ref, ws_ref, bs_ref, bl_ref)

    @pl.when(pl.program_id(0) == 0)
    def _():
        sl_ref[...] = jnp.zeros((1, 1), jnp.float32)

    d = e1 - e2
    sl_ref[...] += jnp.sum(d * d).reshape(1, 1)
    ef_ref[...] = (0.5 * (e1 + e2)).astype(jnp.bfloat16)


def _edge1_mlp(g, w0, ws, bs, bl):
    return pl.pallas_call(
        _edge1_body,
        grid=(N_BLKS,),
        in_specs=[
            pl.BlockSpec((E_BLK, D), lambda i: (i, 0)),
            pl.BlockSpec((E_BLK, D), lambda i: (i + N_BLKS, 0)),
            pl.BlockSpec((2 * D, DH), lambda i: (0, 0)),
            pl.BlockSpec((5, DH, DH), lambda i: (0, 0, 0)),
            pl.BlockSpec((5, DH), lambda i: (0, 0)),
            pl.BlockSpec((1, DH), lambda i: (0, 0)),
        ],
        out_specs=[
            pl.BlockSpec((E_BLK, DH), lambda i: (i, 0)),
            pl.BlockSpec((1, 1), lambda i: (0, 0)),
        ],
        out_shape=[
            jax.ShapeDtypeStruct((NE, DH), jnp.bfloat16),
            jax.ShapeDtypeStruct((1, 1), jnp.float32),
        ],
    )(g, g, w0, ws, bs, bl)


def _edge2_body(gs_ref, gd_ref, ef1_ref, w0_ref, ws_ref, bs_ref, bl_ref,
                wfe_ref, wfp_ref, fb_ref, ef_ref, sl_ref):
    e1, e2 = _edge_stack(gs_ref, gd_ref, w0_ref, ws_ref, bs_ref, bl_ref)

    @pl.when(pl.program_id(0) == 0)
    def _():
        sl_ref[...] = jnp.zeros((1, 1), jnp.float32)

    d = e1 - e2
    sl_ref[...] += jnp.sum(d * d).reshape(1, 1)
    e = (0.5 * (e1 + e2)).astype(jnp.bfloat16)
    o = jnp.dot(e, wfe_ref[...], preferred_element_type=jnp.float32)
    o += jnp.dot(ef1_ref[...], wfp_ref[...], preferred_element_type=jnp.float32)
    ef_ref[...] = o + fb_ref[...]


def _edge2_mlp(g, ef1, w0, ws, bs, bl, wfe, wfp, fb):
    return pl.pallas_call(
        _edge2_body,
        grid=(N_BLKS,),
        in_specs=[
            pl.BlockSpec((E_BLK, D), lambda i: (i, 0)),
            pl.BlockSpec((E_BLK, D), lambda i: (i + N_BLKS, 0)),
            pl.BlockSpec((E_BLK, DH), lambda i: (i, 0)),
            pl.BlockSpec((2 * D, DH), lambda i: (0, 0)),
            pl.BlockSpec((5, DH, DH), lambda i: (0, 0, 0)),
            pl.BlockSpec((5, DH), lambda i: (0, 0)),
            pl.BlockSpec((1, DH), lambda i: (0, 0)),
            pl.BlockSpec((DH, D), lambda i: (0, 0)),
            pl.BlockSpec((DH, D), lambda i: (0, 0)),
            pl.BlockSpec((1, D), lambda i: (0, 0)),
        ],
        out_specs=[
            pl.BlockSpec((E_BLK, D), lambda i: (i, 0)),
            pl.BlockSpec((1, 1), lambda i: (0, 0)),
        ],
        out_shape=[
            jax.ShapeDtypeStruct((NE, D), jnp.float32),
            jax.ShapeDtypeStruct((1, 1), jnp.float32),
        ],
    )(g, g, ef1, w0, ws, bs, bl, wfe, wfp, fb)


def _stack_mlp(p):
    w0 = p[0][0].astype(jnp.bfloat16)
    ws = jnp.stack([w for (w, _) in p[1:]]).astype(jnp.bfloat16)
    bs = jnp.stack([b for (_, b) in p[:-1]]).astype(jnp.bfloat16)
    bl = p[-1][1].reshape(1, -1)
    return w0, ws, bs, bl


def kernel(node_features, edge_index, angles, gt_edges, params):
    del gt_edges
    src = edge_index[0]
    dst = edge_index[1]
    idx_all = jnp.concatenate([src, dst])
    n_ch = NE // CH
    npw_s = -(-n_ch // NW)
    pad_rows = -(-(n_ch - (NW - 1) * npw_s) // 8) * 8 - (n_ch - (NW - 1) * npw_s)
    dst2 = jnp.concatenate(
        [dst, jnp.zeros((pad_rows * CH,), jnp.int32)]).reshape(-1, CH)
    ang = angles.reshape(NE, 1)

    m1w0, m1ws, m1bs, m1bl = _stack_mlp(params["nc1"]["msg"])
    m2w0, m2ws, m2bs, m2bl = _stack_mlp(params["nc2"]["msg"])
    e1w0, e1ws, e1bs, e1bl = _stack_mlp(params["ec1"]["edge"])
    e2w0, e2ws, e2bs, e2bl = _stack_mlp(params["ec2"]["edge"])
    u1w, u1b = params["nc1"]["upd"][0]
    u2w, u2b = params["nc2"]["upd"][0]
    fw, fb = params["ec2"]["fuse"][0]
    u1w = u1w.astype(jnp.bfloat16)
    u2w = u2w.astype(jnp.bfloat16)
    fw = fw.astype(jnp.bfloat16)
    u1b = u1b.reshape(1, D)
    u2b = u2b.reshape(1, D)
    fb = fb.reshape(1, D)
    wfe, wfp = fw[:DH], fw[DH:]

    zeros_d = jnp.zeros((NN, D), jnp.float32)
    ones_d = jnp.ones((CH, D), jnp.float32)

    x0 = node_features

    dp = _segment_count(dst2, zeros_d, ones_d)

    g0 = _gather_rows(x0, idx_all)
    m1 = _msg_mlp(g0, ang, m1w0, m1ws, m1bs, m1bl)
    a1 = _segment_sum(m1, dst2, zeros_d)
    x1 = _upd_mlp(x0, a1, dp, u1w, u1b)

    g1 = _gather_rows(x1, idx_all)

    m2 = _msg_mlp(g1, ang, m2w0, m2ws, m2bs, m2bl)
    a2 = _segment_sum(m2, dst2, zeros_d)
    x2 = _upd_mlp(x1, a2, dp, u2w, u2b)
    g2 = _gather_rows(x2, idx_all)

    ef1, sl1 = _edge1_mlp(g1, e1w0, e1ws, e1bs, e1bl)

    ef, sl2 = _edge2_mlp(g2, ef1, e2w0, e2ws, e2bs, e2bl, wfe, wfp, fb)

    side_loss = (sl1[0, 0] + sl2[0, 0]) / (2.0 * NE * DH)
    return ef, side_loss


# speedup vs baseline: 1.2185x; 1.0089x over previous
import functools

import jax
import jax.numpy as jnp
from jax import lax
from jax.experimental import pallas as pl
from jax.experimental.pallas import tpu as pltpu
from jax.experimental.pallas import tpu_sc as plsc

NN = 10000
NE = 160000
D = 128
DH = 384

NC = 2
NS = 16
NW = NC * NS
CH = 128

E_BLK = 2000
E_MSG = 4000
R_BLK = 2000

_SC_MESH = plsc.VectorSubcoreMesh(core_axis_name="c", subcore_axis_name="s")


def _gather_rows(table, idx):
    B = idx.shape[0]
    n_chunks = B // CH
    npw = -(-n_chunks // NW)
    last_n = n_chunks - (NW - 1) * npw

    @functools.partial(
        pl.kernel,
        out_type=jax.ShapeDtypeStruct((B, D), jnp.float32),
        mesh=_SC_MESH,
        scratch_types=[
            pltpu.VMEM((npw * CH,), jnp.int32),
            pltpu.VMEM((CH, D), jnp.float32),
            pltpu.VMEM((CH, D), jnp.float32),
            pltpu.SemaphoreType.DMA,
            pltpu.SemaphoreType.DMA,
            pltpu.SemaphoreType.DMA,
        ],
    )
    def k(table_hbm, idx_hbm, out_hbm, idx_v, r0_v, r1_v, gsem, w0sem, w1sem):
        wid = lax.axis_index("s") * NC + lax.axis_index("c")
        base = wid * npw
        my_n = jnp.where(wid < NW - 1, npw, last_n)
        rows = (r0_v, r1_v)
        wsems = (w0sem, w1sem)

        @pl.when(wid < NW - 1)
        def _():
            pltpu.sync_copy(idx_hbm.at[pl.ds(base * CH, npw * CH)], idx_v)

        @pl.when(wid == NW - 1)
        def _():
            pltpu.sync_copy(idx_hbm.at[pl.ds(base * CH, last_n * CH)],
                            idx_v.at[pl.ds(0, last_n * CH)])

        @pl.loop(0, npw, step=2)
        def _(t0):
            for b in range(2):
                t = t0 + b

                @pl.when((t >= 2) & (t < my_n))
                def _():
                    pltpu.make_async_copy(
                        rows[b], out_hbm.at[pl.ds(0, CH)], wsems[b]).wait()

            for b in range(2):
                t = t0 + b

                @pl.when(t < my_n)
                def _():
                    pltpu.async_copy(
                        table_hbm.at[idx_v.at[pl.ds(t * CH, CH)]],
                        rows[b], gsem)

            for b in range(2):
                t = t0 + b

                @pl.when(t < my_n)
                def _():
                    pltpu.make_async_copy(
                        table_hbm.at[idx_v.at[pl.ds(t * CH, CH)]],
                        rows[b], gsem).wait()

            for b in range(2):
                t = t0 + b

                @pl.when(t < my_n)
                def _():
                    pltpu.async_copy(
                        rows[b], out_hbm.at[pl.ds((base + t) * CH, CH)],
                        wsems[b])

        @pl.when(my_n >= 2)
        def _():
            pltpu.make_async_copy(r0_v, out_hbm.at[pl.ds(0, CH)], w0sem).wait()
            pltpu.make_async_copy(r1_v, out_hbm.at[pl.ds(0, CH)], w1sem).wait()

        @pl.when(my_n == 1)
        def _():
            pltpu.make_async_copy(r0_v, out_hbm.at[pl.ds(0, CH)], w0sem).wait()

    return k(table, idx)


def _per_sub_slices(sid, fn):
    rps = 624

    @pl.when(sid < NS - 1)
    def _():
        fn(sid * rps, rps)

    @pl.when(sid == NS - 1)
    def _():
        fn((NS - 1) * rps, NN - (NS - 1) * rps)


def _segment_sum(m, dst2, zeros_d):
    n_chunks = m.shape[0] // CH
    npw = -(--(-n_chunks // NW) // 8) * 8

    @functools.partial(
        pl.kernel,
        out_type=jax.ShapeDtypeStruct((NC, NN, D), jnp.float32),
        mesh=_SC_MESH,
        scratch_types=[
            pltpu.VMEM((npw, CH), jnp.int32),
            pltpu.VMEM((CH, D), jnp.float32),
            pltpu.VMEM((CH, D), jnp.float32),
            pltpu.VMEM_SHARED((NN, D), jnp.float32),
            pltpu.SemaphoreType.DMA,
            pltpu.SemaphoreType.DMA,
        ],
    )
    def k(m_hbm, dst_hbm, z_d_hbm, agg_hbm, idx_v, m0_v, m1_v, acc_sh,
          l0sem, l1sem):
        cid = lax.axis_index("c")
        sid = lax.axis_index("s")
        wid = sid * NC + cid
        base = wid * npw
        my_n = jnp.clip(n_chunks - base, 0, npw)
        bufs = (m0_v, m1_v)
        sems = (l0sem, l1sem)

        _per_sub_slices(sid, lambda o, n: pltpu.sync_copy(
            z_d_hbm.at[pl.ds(o, n)], acc_sh.at[pl.ds(o, n)]))

        @pl.when(my_n > 0)
        def _():
            pltpu.sync_copy(dst_hbm.at[pl.ds(base, npw)], idx_v)

        plsc.subcore_barrier()

        @pl.loop(0, npw, step=2)
        def _(t0):
            for b in range(2):
                t = t0 + b

                @pl.when(t < my_n)
                def _():
                    pltpu.async_copy(m_hbm.at[pl.ds((base + t) * CH, CH)],
                                     bufs[b], sems[b])

            for b in range(2):
                t = t0 + b

                @pl.when(t < my_n)
                def _():
                    pltpu.make_async_copy(
                        m_hbm.at[pl.ds((base + t) * CH, CH)],
                        bufs[b], sems[b]).wait()
                    pltpu.sync_copy(bufs[b], acc_sh.at[idx_v.at[t]],
                                    add=True)

        plsc.subcore_barrier()
        _per_sub_slices(sid, lambda o, n: pltpu.sync_copy(
            acc_sh.at[pl.ds(o, n)], agg_hbm.at[cid, pl.ds(o, n)]))

    return k(m, dst2, zeros_d)


def _segment_count(dst2, zeros_d, ones_d):
    n_chunks = NE // CH
    npw = -(-n_chunks // NW)
    last_n = n_chunks - (NW - 1) * npw

    @functools.partial(
        pl.kernel,
        out_type=jax.ShapeDtypeStruct((NC, NN, D), jnp.float32),
        mesh=_SC_MESH,
        scratch_types=[
            pltpu.VMEM((npw, CH), jnp.int32),
            pltpu.VMEM((CH, D), jnp.float32),
            pltpu.VMEM_SHARED((NN, D), jnp.float32),
            pltpu.SemaphoreType.DMA,
        ],
    )
    def k(dst_hbm, z_d_hbm, ones_hbm, deg_hbm, idx_v, ones_v, acc_sh, sem):
        cid = lax.axis_index("c")
        sid = lax.axis_index("s")
        wid = sid * NC + cid
        base = wid * npw
        my_n = jnp.where(wid < NW - 1, npw, last_n)

        _per_sub_slices(sid, lambda o, n: pltpu.sync_copy(
            z_d_hbm.at[pl.ds(o, n)], acc_sh.at[pl.ds(o, n)]))
        pltpu.sync_copy(ones_hbm, ones_v)

        @pl.when(wid < NW - 1)
        def _():
            pltpu.sync_copy(dst_hbm.at[pl.ds(base, npw)], idx_v)

        last_ld = -(-last_n // 8) * 8

        @pl.when(wid == NW - 1)
        def _():
            pltpu.sync_copy(dst_hbm.at[pl.ds(base, last_ld)],
                            idx_v.at[pl.ds(0, last_ld)])

        plsc.subcore_barrier()

        @pl.loop(0, npw)
        def _(t):
            @pl.when(t < my_n)
            def _():
                pltpu.sync_copy(ones_v, acc_sh.at[idx_v.at[t]], add=True)

        plsc.subcore_barrier()
        _per_sub_slices(sid, lambda o, n: pltpu.sync_copy(
            acc_sh.at[pl.ds(o, n)], deg_hbm.at[cid, pl.ds(o, n)]))

    return k(dst2, zeros_d, ones_d)


def _chain(a, ws_ref, bs_ref, bs_f32, n_hidden):
    for i in range(n_hidden - 1):
        a = jnp.dot(a, ws_ref[i], preferred_element_type=jnp.float32)
        a = jnp.maximum(a.astype(jnp.bfloat16) + bs_ref[i + 1],
                        jnp.bfloat16(0.0))
    a = jnp.dot(a, ws_ref[n_hidden - 1], preferred_element_type=jnp.float32)
    return a + bs_f32


def _msg_body(gs_ref, gd_ref, ang_ref, w0_ref, ws_ref, bs_ref, bl_ref, m_ref):
    h = jnp.concatenate([gs_ref[...], gd_ref[...]], axis=1)
    a = jnp.dot(h, w0_ref[...], preferred_element_type=jnp.float32)
    a = jnp.maximum(a.astype(jnp.bfloat16) + bs_ref[0], jnp.bfloat16(0.0))
    a = _chain(a, ws_ref, bs_ref, bl_ref[...], 5)
    m_ref[...] = a * ang_ref[...]


N_BLKS = NE // E_BLK
N_BLKS_MSG = NE // E_MSG


def _msg_mlp(g, ang, w0, ws, bs, bl, half=None):
    if half is None:
        nb, b0, gdo = N_BLKS_MSG, 0, NE // E_MSG
    else:
        nb = NE // (2 * E_MSG)
        b0 = half * nb
        gdo = NE // E_MSG if g.shape[0] == 2 * NE else NE // (2 * E_MSG)
    n_out = nb * E_MSG
    return pl.pallas_call(
        _msg_body,
        grid=(nb,),
        in_specs=[
            pl.BlockSpec((E_MSG, D), lambda i: (i + b0, 0)),
            pl.BlockSpec((E_MSG, D), lambda i: (i + b0 + gdo, 0)),
            pl.BlockSpec((E_MSG, 1), lambda i: (i + b0, 0)),
            pl.BlockSpec((2 * D, D), lambda i: (0, 0)),
            pl.BlockSpec((5, D, D), lambda i: (0, 0, 0)),
            pl.BlockSpec((5, D), lambda i: (0, 0)),
            pl.BlockSpec((1, D), lambda i: (0, 0)),
        ],
        out_specs=pl.BlockSpec((E_MSG, D), lambda i: (i, 0)),
        out_shape=jax.ShapeDtypeStruct((n_out, D), jnp.float32),
    )(g, g, ang, w0, ws, bs, bl)


def _upd_body(x_ref, agg_ref, agg2_ref, deg_ref, w_ref, b_ref, o_ref):
    agg = (agg_ref[0] + agg_ref[1]) + (agg2_ref[0] + agg2_ref[1])
    deg = deg_ref[0, :, 0:1] + deg_ref[1, :, 0:1]
    agg = agg / jnp.maximum(deg, 1.0)
    h = jnp.concatenate([x_ref[...], agg], axis=1).astype(jnp.bfloat16)
    o = jnp.dot(h, w_ref[...], preferred_element_type=jnp.float32)
    o = o + b_ref[...]
    o_ref[...] = jnp.maximum(o, 0.0)


def _upd_mlp(x, agg_a, agg_b, deg_p, w, b):
    grid = NN // R_BLK
    return pl.pallas_call(
        _upd_body,
        grid=(grid,),
        in_specs=[
            pl.BlockSpec((R_BLK, D), lambda i: (i, 0)),
            pl.BlockSpec((NC, R_BLK, D), lambda i: (0, i, 0)),
            pl.BlockSpec((NC, R_BLK, D), lambda i: (0, i, 0)),
            pl.BlockSpec((NC, R_BLK, D), lambda i: (0, i, 0)),
            pl.BlockSpec((2 * D, D), lambda i: (0, 0)),
            pl.BlockSpec((1, D), lambda i: (0, 0)),
        ],
        out_specs=pl.BlockSpec((R_BLK, D), lambda i: (i, 0)),
        out_shape=jax.ShapeDtypeStruct((NN, D), jnp.float32),
    )(x, agg_a, agg_b, deg_p, w, b)


def _edge_stack(gs_ref, gd_ref, w0_ref, ws_ref, bs_ref, bl_ref):
    hf = jnp.concatenate([gs_ref[...], gd_ref[...]], axis=1)
    hr = jnp.concatenate([gd_ref[...], gs_ref[...]], axis=1)
    h = jnp.concatenate([hf, hr], axis=0)
    a = jnp.dot(h, w0_ref[...], preferred_element_type=jnp.float32)
    a = jnp.maximum(a.astype(jnp.bfloat16) + bs_ref[0], jnp.bfloat16(0.0))
    a = _chain(a, ws_ref, bs_ref, bl_ref[...], 5)
    return a[:E_BLK], a[E_BLK:]


def _edge1_body(gs_ref, gd_ref, w0_ref, ws_ref, bs_ref, bl_ref, ef_ref, sl_ref):
    e1, e2 = _edge_stack(gs_ref, gd_ref, w0_ref, ws_ref, bs_ref, bl_ref)

    @pl.when(pl.program_id(0) == 0)
    def _():
        sl_ref[...] = jnp.zeros((1, 1), jnp.float32)

    d = e1 - e2
    sl_ref[...] += jnp.sum(d * d).reshape(1, 1)
    ef_ref[...] = (0.5 * (e1 + e2)).astype(jnp.bfloat16)


def _edge1_mlp(g, w0, ws, bs, bl):
    return pl.pallas_call(
        _edge1_body,
        grid=(N_BLKS,),
        in_specs=[
            pl.BlockSpec((E_BLK, D), lambda i: (i, 0)),
            pl.BlockSpec((E_BLK, D), lambda i: (i + N_BLKS, 0)),
            pl.BlockSpec((2 * D, DH), lambda i: (0, 0)),
            pl.BlockSpec((5, DH, DH), lambda i: (0, 0, 0)),
            pl.BlockSpec((5, DH), lambda i: (0, 0)),
            pl.BlockSpec((1, DH), lambda i: (0, 0)),
        ],
        out_specs=[
            pl.BlockSpec((E_BLK, DH), lambda i: (i, 0)),
            pl.BlockSpec((1, 1), lambda i: (0, 0)),
        ],
        out_shape=[
            jax.ShapeDtypeStruct((NE, DH), jnp.bfloat16),
            jax.ShapeDtypeStruct((1, 1), jnp.float32),
        ],
    )(g, g, w0, ws, bs, bl)


def _edge2_body(gs_ref, gd_ref, ef1_ref, w0_ref, ws_ref, bs_ref, bl_ref,
                wfe_ref, wfp_ref, fb_ref, ef_ref, sl_ref):
    e1, e2 = _edge_stack(gs_ref, gd_ref, w0_ref, ws_ref, bs_ref, bl_ref)

    @pl.when(pl.program_id(0) == 0)
    def _():
        sl_ref[...] = jnp.zeros((1, 1), jnp.float32)

    d = e1 - e2
    sl_ref[...] += jnp.sum(d * d).reshape(1, 1)
    e = (0.5 * (e1 + e2)).astype(jnp.bfloat16)
    o = jnp.dot(e, wfe_ref[...], preferred_element_type=jnp.float32)
    o += jnp.dot(ef1_ref[...], wfp_ref[...], preferred_element_type=jnp.float32)
    ef_ref[...] = o + fb_ref[...]


def _edge2_mlp(g, ef1, w0, ws, bs, bl, wfe, wfp, fb):
    return pl.pallas_call(
        _edge2_body,
        grid=(N_BLKS,),
        in_specs=[
            pl.BlockSpec((E_BLK, D), lambda i: (i, 0)),
            pl.BlockSpec((E_BLK, D), lambda i: (i + N_BLKS, 0)),
            pl.BlockSpec((E_BLK, DH), lambda i: (i, 0)),
            pl.BlockSpec((2 * D, DH), lambda i: (0, 0)),
            pl.BlockSpec((5, DH, DH), lambda i: (0, 0, 0)),
            pl.BlockSpec((5, DH), lambda i: (0, 0)),
            pl.BlockSpec((1, DH), lambda i: (0, 0)),
            pl.BlockSpec((DH, D), lambda i: (0, 0)),
            pl.BlockSpec((DH, D), lambda i: (0, 0)),
            pl.BlockSpec((1, D), lambda i: (0, 0)),
        ],
        out_specs=[
            pl.BlockSpec((E_BLK, D), lambda i: (i, 0)),
            pl.BlockSpec((1, 1), lambda i: (0, 0)),
        ],
        out_shape=[
            jax.ShapeDtypeStruct((NE, D), jnp.float32),
            jax.ShapeDtypeStruct((1, 1), jnp.float32),
        ],
    )(g, g, ef1, w0, ws, bs, bl, wfe, wfp, fb)


def _stack_mlp(p):
    w0 = p[0][0].astype(jnp.bfloat16)
    ws = jnp.stack([w for (w, _) in p[1:]]).astype(jnp.bfloat16)
    bs = jnp.stack([b for (_, b) in p[:-1]]).astype(jnp.bfloat16)
    bl = p[-1][1].reshape(1, -1)
    return w0, ws, bs, bl


def kernel(node_features, edge_index, angles, gt_edges, params):
    del gt_edges
    src = edge_index[0]
    dst = edge_index[1]
    idx_all = jnp.concatenate([src, dst])
    NHC = NE // (2 * CH)
    npw_h = -(--(-NHC // NW) // 8) * 8
    lim_h = -(-NHC // npw_h) * npw_h
    pad = NHC + lim_h - (NE // CH)
    dst2 = jnp.concatenate(
        [dst, jnp.zeros((pad * CH,), jnp.int32)]).reshape(-1, CH)
    dstA = dst2[:lim_h]
    dstB = dst2[NHC:NHC + lim_h]
    ang = angles.reshape(NE, 1)

    m1w0, m1ws, m1bs, m1bl = _stack_mlp(params["nc1"]["msg"])
    m2w0, m2ws, m2bs, m2bl = _stack_mlp(params["nc2"]["msg"])
    e1w0, e1ws, e1bs, e1bl = _stack_mlp(params["ec1"]["edge"])
    e2w0, e2ws, e2bs, e2bl = _stack_mlp(params["ec2"]["edge"])
    u1w, u1b = params["nc1"]["upd"][0]
    u2w, u2b = params["nc2"]["upd"][0]
    fw, fb = params["ec2"]["fuse"][0]
    u1w = u1w.astype(jnp.bfloat16)
    u2w = u2w.astype(jnp.bfloat16)
    fw = fw.astype(jnp.bfloat16)
    u1b = u1b.reshape(1, D)
    u2b = u2b.reshape(1, D)
    fb = fb.reshape(1, D)
    wfe, wfp = fw[:DH], fw[DH:]

    zeros_d = jnp.zeros((NN, D), jnp.float32)
    ones_d = jnp.ones((CH, D), jnp.float32)

    x0 = node_features

    dp = _segment_count(dst2, zeros_d, ones_d)

    H = NE // 2
    idxA = jnp.concatenate([src[:H], dst[:H]])
    idxB = jnp.concatenate([src[H:], dst[H:]])
    g0a = _gather_rows(x0, idxA)
    g0b = _gather_rows(x0, idxB)
    m1a = _msg_mlp(g0a, ang, m1w0, m1ws, m1bs, m1bl, half=0)
    m1b = _msg_mlp(g0b, ang[H:], m1w0, m1ws, m1bs, m1bl, half=0)
    a1a = _segment_sum(m1a, dstA, zeros_d)
    a1b = _segment_sum(m1b, dstB, zeros_d)
    x1 = _upd_mlp(x0, a1a, a1b, dp, u1w, u1b)

    g1 = _gather_rows(x1, idx_all)

    m2a = _msg_mlp(g1, ang, m2w0, m2ws, m2bs, m2bl, half=0)
    m2b = _msg_mlp(g1, ang, m2w0, m2ws, m2bs, m2bl, half=1)
    ef1, sl1 = _edge1_mlp(g1, e1w0, e1ws, e1bs, e1bl)
    a2a = _segment_sum(m2a, dstA, zeros_d)
    a2b = _segment_sum(m2b, dstB, zeros_d)
    x2 = _upd_mlp(x1, a2a, a2b, dp, u2w, u2b)
    g2 = _gather_rows(x2, idx_all)

    ef, sl2 = _edge2_mlp(g2, ef1, e2w0, e2ws, e2bs, e2bl, wfe, wfp, fb)

    side_loss = (sl1[0, 0] + sl2[0, 0]) / (2.0 * NE * DH)
    return ef, side_loss
